# Initial kernel scaffold; baseline (speedup 1.0000x reference)
import jax
import jax.numpy as jnp
from jax.experimental import pallas as pl


def kernel(pos, y, batch, emb_W, ce_W1, ce_b1, ce_W2, ce_b2, cw1_0, cb1_0, cw2_0, cb2_0, nw_0, nb_0, cw1_1, cb1_1, cw2_1, cb2_1, nw_1, nb_1, cw1_2, cb1_2, cw2_2, cb2_2, nw_2, nb_2, fw1, fb1, fnw1, fnb1, fw2, fb2, fnw2, fnb2, fw3, fb3):
    raise NotImplementedError("write your pallas kernel here")


# speedup vs baseline: 5.1144x; 5.1144x over previous
import functools

import jax
import jax.numpy as jnp
from jax import lax
from jax.experimental import pallas as pl
from jax.experimental.pallas import tpu as pltpu
from jax.experimental.pallas import tpu_sc as plsc

N = 10000
NPAD = 10240
B = 8
K = 16
R = 256
NBLK = NPAD // R
BIG = 1e30
BIG2 = 2e30

SC_NC = 2
SC_NS = 16
SC_NW = SC_NC * SC_NS


def _dotH(a, b):
    return jnp.dot(a, b, precision=jax.lax.Precision.HIGHEST,
                   preferred_element_type=jnp.float32)


def _dotD(a, b):
    return jnp.dot(a, b, precision=jax.lax.Precision.DEFAULT,
                   preferred_element_type=jnp.float32)


def _elu(x):
    return jnp.where(x > 0, x, jnp.exp(jnp.minimum(x, 0.0)) - 1.0)


def _prep_body(y_ref, bc_ref, emb_ref, w1_ref, b1_ref, w2_ref, b2_ref,
               c_ref, cb_ref):
    y = y_ref[...]
    oh = (y == lax.broadcasted_iota(jnp.int32, (1, 16), 1)).astype(jnp.float32)
    c = _elu(_dotH(oh, emb_ref[...]))
    c = _elu(_dotD(c, w1_ref[...]) + b1_ref[...])
    c = _dotD(c, w2_ref[...]) + b2_ref[...]
    c_ref[...] = c
    bc = bc_ref[...]
    m = (bc == lax.broadcasted_iota(jnp.int32, (1, B), 1)).astype(jnp.float32)
    cb_ref[...] = _dotH(m, c)


def _prep(y2, batch_col, emb_W, ce_W1, ce_b1, ce_W2, ce_b2):
    return pl.pallas_call(
        _prep_body,
        out_shape=(
            jax.ShapeDtypeStruct((B, 128), jnp.float32),
            jax.ShapeDtypeStruct((NPAD, 128), jnp.float32),
        ),
    )(y2, batch_col, emb_W, ce_W1, ce_b1, ce_W2, ce_b2)


def _knn_body(x_ref, xt_ref, bc_ref, br_ref, idx_ref):
    blk = pl.program_id(0)
    xb = x_ref[...]
    xt = xt_ref[...]
    p = _dotD(xb, xt)
    sqj = jnp.sum(xt * xt, axis=0, keepdims=True)
    sqi = jnp.sum(xb * xb, axis=1, keepdims=True)
    score = (sqi - 2.0 * p) + sqj
    bc = bc_ref[...]
    br = br_ref[...]
    rowf = (blk * R + lax.broadcasted_iota(jnp.int32, (R, 1), 0)).astype(
        jnp.float32)
    colf = lax.broadcasted_iota(jnp.int32, (1, NPAD), 1).astype(jnp.float32)
    ok = (bc == br) & (bc < B) & (rowf != colf)
    score = jnp.where(ok, score, BIG)
    for t in range(K):
        m = jnp.min(score, axis=1, keepdims=True)
        cand = jnp.where(score == m, colf, float(NPAD))
        j = jnp.min(cand, axis=1, keepdims=True)
        j = jnp.minimum(j, float(NPAD - 1))
        idx_ref[:, t:t + 1] = j.astype(jnp.int32)
        score = jnp.where(colf == j, BIG2, score)


def _knn(xin, xint, batch_col, batch_row, d):
    return pl.pallas_call(
        _knn_body,
        grid=(NBLK,),
        in_specs=[
            pl.BlockSpec((R, d), lambda i: (i, 0)),
            pl.BlockSpec((d, NPAD), lambda i: (0, 0)),
            pl.BlockSpec((R, 1), lambda i: (i, 0)),
            pl.BlockSpec((1, NPAD), lambda i: (0, 0)),
        ],
        out_specs=pl.BlockSpec((R, K), lambda i: (i, 0)),
        out_shape=jax.ShapeDtypeStruct((NPAD, K), jnp.int32),
    )(xin, xint, batch_col, batch_row)


def _sc_gather(table, idx_flat, h):
    nk = NPAD * K
    per_w = nk // SC_NW
    chunk = 64
    n_iter = per_w // chunk
    mesh = plsc.VectorSubcoreMesh(core_axis_name="c", subcore_axis_name="s")

    @functools.partial(
        pl.kernel,
        mesh=mesh,
        out_type=jax.ShapeDtypeStruct((nk, h), jnp.float32),
        scratch_types=[
            pltpu.VMEM((chunk,), jnp.int32),
            pltpu.VMEM((chunk, h), jnp.float32),
            pltpu.SemaphoreType.DMA,
        ],
    )
    def gk(table_hbm, idx_hbm, out_hbm, idx_v, rows_v, sem):
        wid = lax.axis_index("s") * SC_NC + lax.axis_index("c")
        base = wid * per_w

        def body(it, carry):
            off = base + it * chunk
            pltpu.sync_copy(idx_hbm.at[pl.ds(off, chunk)], idx_v)
            pltpu.async_copy(table_hbm.at[idx_v], rows_v, sem).wait()
            pltpu.sync_copy(rows_v, out_hbm.at[pl.ds(off, chunk)])
            return carry

        lax.fori_loop(0, n_iter, body, 0)

    return gk(table, idx_flat)


def _conv_body(x_ref, g_ref, bc_ref, w1_ref, b1_ref, w2_ref, b2_ref,
               o_ref, st_ref):
    xb = x_ref[...]
    w1 = w1_ref[...]
    w2 = w2_ref[...]
    b1 = b1_ref[...]
    acc = None
    for k in range(K):
        xj = g_ref[:, k, :]
        e = jnp.concatenate([xb, xj - xb], axis=1)
        h = _elu(_dotD(e, w1) + b1)
        p = _dotD(h, w2)
        acc = p if acc is None else jnp.maximum(acc, p)
    o = acc + b2_ref[...]
    o_ref[...] = o
    bc = bc_ref[...]
    m8 = (bc == lax.broadcasted_iota(jnp.int32, (1, B), 1)).astype(jnp.float32)
    cnt = jnp.sum(m8, axis=0, keepdims=True)
    s1 = jnp.sum(m8 * jnp.sum(o, axis=1, keepdims=True), axis=0, keepdims=True)
    part = jnp.concatenate([cnt, s1, jnp.zeros((6, B), jnp.float32)], axis=0)

    @pl.when(pl.program_id(0) == 0)
    def _():
        st_ref[...] = part

    @pl.when(pl.program_id(0) != 0)
    def _():
        st_ref[...] += part


def _conv(xin, g3, batch_col, w1, b1, w2, b2, d, h, o):
    return pl.pallas_call(
        _conv_body,
        grid=(NBLK,),
        in_specs=[
            pl.BlockSpec((R, d), lambda i: (i, 0)),
            pl.BlockSpec((R, K, d), lambda i: (i, 0, 0)),
            pl.BlockSpec((R, 1), lambda i: (i, 0)),
            pl.BlockSpec((2 * d, h), lambda i: (0, 0)),
            pl.BlockSpec((1, h), lambda i: (0, 0)),
            pl.BlockSpec((h, o), lambda i: (0, 0)),
            pl.BlockSpec((1, o), lambda i: (0, 0)),
        ],
        out_specs=(
            pl.BlockSpec((R, o), lambda i: (i, 0)),
            pl.BlockSpec((8, B), lambda i: (0, 0)),
        ),
        out_shape=(
            jax.ShapeDtypeStruct((NPAD, o), jnp.float32),
            jax.ShapeDtypeStruct((8, B), jnp.float32),
        ),
    )(xin, g3, batch_col, w1, b1, w2, b2)


def _stats2_body(x_ref, bc_ref, st1_ref, st2_ref):
    x = x_ref[...]
    f = x.shape[1]
    bc = bc_ref[...]
    cnt = st1_ref[0:1, :]
    s1 = st1_ref[1:2, :]
    denom = jnp.maximum(cnt, 1.0) * float(f)
    mean = s1 / denom
    m8 = (bc == lax.broadcasted_iota(jnp.int32, (1, B), 1)).astype(jnp.float32)
    mean_b = jnp.sum(m8 * mean, axis=1, keepdims=True)
    xc = x - mean_b
    s2 = jnp.sum(m8 * jnp.sum(xc * xc, axis=1, keepdims=True), axis=0,
                 keepdims=True)
    part = jnp.concatenate([s2, jnp.zeros((7, B), jnp.float32)], axis=0)

    @pl.when(pl.program_id(0) == 0)
    def _():
        st2_ref[...] = part

    @pl.when(pl.program_id(0) != 0)
    def _():
        st2_ref[...] += part


def _stats2(x, batch_col, st1, f):
    return pl.pallas_call(
        _stats2_body,
        grid=(NBLK,),
        in_specs=[
            pl.BlockSpec((R, f), lambda i: (i, 0)),
            pl.BlockSpec((R, 1), lambda i: (i, 0)),
            pl.BlockSpec((8, B), lambda i: (0, 0)),
        ],
        out_specs=pl.BlockSpec((8, B), lambda i: (0, 0)),
        out_shape=jax.ShapeDtypeStruct((8, B), jnp.float32),
    )(x, batch_col, st1)


def _ln_body(x_ref, bc_ref, st1_ref, st2_ref, w_ref, b_ref, o_ref):
    x = x_ref[...]
    f = x.shape[1]
    bc = bc_ref[...]
    cnt = st1_ref[0:1, :]
    s1 = st1_ref[1:2, :]
    s2 = st2_ref[0:1, :]
    denom = jnp.maximum(cnt, 1.0) * float(f)
    mean = s1 / denom
    var = s2 / denom
    m8 = (bc == lax.broadcasted_iota(jnp.int32, (1, B), 1)).astype(jnp.float32)
    mean_b = jnp.sum(m8 * mean, axis=1, keepdims=True)
    var_b = jnp.sum(m8 * var, axis=1, keepdims=True)
    out = (x - mean_b) * lax.rsqrt(var_b + 1e-5) * w_ref[...] + b_ref[...]
    out = _elu(out)
    o_ref[...] = jnp.where(bc < B, out, 0.0)


def _graph_ln(x, batch_col, st1, st2, w, b, f):
    return pl.pallas_call(
        _ln_body,
        grid=(NBLK,),
        in_specs=[
            pl.BlockSpec((R, f), lambda i: (i, 0)),
            pl.BlockSpec((R, 1), lambda i: (i, 0)),
            pl.BlockSpec((8, B), lambda i: (0, 0)),
            pl.BlockSpec((8, B), lambda i: (0, 0)),
            pl.BlockSpec((1, f), lambda i: (0, 0)),
            pl.BlockSpec((1, f), lambda i: (0, 0)),
        ],
        out_specs=pl.BlockSpec((R, f), lambda i: (i, 0)),
        out_shape=jax.ShapeDtypeStruct((NPAD, f), jnp.float32),
    )(x, batch_col, st1, st2, w, b)


def _head_body(x_ref, br_ref, c_ref, w1a_ref, w1b_ref, b1_ref, nw1_ref,
               nb1_ref, w2_ref, b2_ref, nw2_ref, nb2_ref, w3_ref, b3_ref,
               o_ref):
    br = br_ref[...]
    m8t = (lax.broadcasted_iota(jnp.int32, (B, 1), 0) == br).astype(
        jnp.float32)
    cnt = jnp.sum(m8t, axis=1, keepdims=True)
    pooled = _dotH(m8t, x_ref[...]) / jnp.maximum(cnt, 1.0)

    def ln(v, w, bb):
        mu = jnp.mean(v, axis=-1, keepdims=True)
        va = jnp.mean((v - mu) ** 2, axis=-1, keepdims=True)
        return (v - mu) * lax.rsqrt(va + 1e-5) * w + bb

    h = (_dotD(pooled, w1a_ref[...]) + _dotD(c_ref[...], w1b_ref[...])
         + b1_ref[...])
    h = _elu(ln(h, nw1_ref[...], nb1_ref[...]))
    h = _dotD(h, w2_ref[...]) + b2_ref[...]
    h = _elu(ln(h, nw2_ref[...], nb2_ref[...]))
    o_ref[...] = _dotD(h, w3_ref[...]) + b3_ref[...]


def _head(x, batch_row, c, fw1a, fw1b, fb1, fnw1, fnb1, fw2, fb2, fnw2,
          fnb2, fw3, fb3):
    return pl.pallas_call(
        _head_body,
        out_shape=jax.ShapeDtypeStruct((B, 1), jnp.float32),
    )(x, batch_row, c, fw1a, fw1b, fb1, fnw1, fnb1, fw2, fb2, fnw2, fnb2,
      fw3, fb3)


def kernel(pos, y, batch, emb_W, ce_W1, ce_b1, ce_W2, ce_b2,
           cw1_0, cb1_0, cw2_0, cb2_0, nw_0, nb_0,
           cw1_1, cb1_1, cw2_1, cb2_1, nw_1, nb_1,
           cw1_2, cb1_2, cw2_2, cb2_2, nw_2, nb_2,
           fw1, fb1, fnw1, fnb1, fw2, fb2, fnw2, fnb2, fw3, fb3):
    batch = batch.astype(jnp.int32)
    batch_p = jnp.pad(batch, (0, NPAD - N), constant_values=B)
    batch_col = batch_p.reshape(NPAD, 1)
    batch_row = batch_p.reshape(1, NPAD)
    y2 = y.astype(jnp.int32).reshape(B, 1)

    c, cb = _prep(y2, batch_col, emb_W, ce_W1, ce_b1, ce_W2, ce_b2)

    hc = [3, 128, 256, 512]
    conv_params = [(cw1_0, cb1_0, cw2_0, cb2_0),
                   (cw1_1, cb1_1, cw2_1, cb2_1),
                   (cw1_2, cb1_2, cw2_2, cb2_2)]
    norm_params = [(nw_0, nb_0), (nw_1, nb_1), (nw_2, nb_2)]

    x = jnp.pad(pos, ((0, NPAD - N), (0, 0)))
    for i in range(3):
        in_c = hc[i] + 128
        hid = hc[i] * 2
        out = hc[i + 1]
        w1, b1, w2, b2 = conv_params[i]
        nw, nb = norm_params[i]
        dp = max(256, ((in_c + 127) // 128) * 128)
        hp = max(128, ((hid + 127) // 128) * 128)

        xin = jnp.concatenate([x, cb], axis=1)
        if xin.shape[1] < dp:
            xin = jnp.pad(xin, ((0, 0), (0, dp - xin.shape[1])))
        w1a = jnp.pad(w1[:in_c], ((0, dp - in_c), (0, 0)))
        w1b = jnp.pad(w1[in_c:], ((0, dp - in_c), (0, 0)))
        w1p = jnp.concatenate([w1a, w1b], axis=0)
        if hid < hp:
            w1p = jnp.pad(w1p, ((0, 0), (0, hp - hid)))
            b1p = jnp.pad(b1, (0, hp - hid))
            w2p = jnp.pad(w2, ((0, hp - hid), (0, 0)))
        else:
            b1p = b1
            w2p = w2

        idx = _knn(xin, xin.T, batch_col, batch_row, dp)
        g = _sc_gather(xin, idx.reshape(NPAD * K), dp)
        xconv, st1 = _conv(xin, g.reshape(NPAD, K, dp), batch_col, w1p,
                           b1p.reshape(1, hp), w2p, b2.reshape(1, out),
                           dp, hp, out)
        st2 = _stats2(xconv, batch_col, st1, out)
        x = _graph_ln(xconv, batch_col, st1, st2, nw.reshape(1, out),
                      nb.reshape(1, out), out)

    fw1a = fw1[:hc[-1]]
    fw1b = fw1[hc[-1]:]
    return _head(x, batch_row, c, fw1a, fw1b, fb1.reshape(1, -1),
                 fnw1.reshape(1, -1), fnb1.reshape(1, -1), fw2,
                 fb2.reshape(1, -1), fnw2.reshape(1, -1), fnb2.reshape(1, -1),
                 fw3, fb3.reshape(1, -1))


# speedup vs baseline: 8.7602x; 1.7129x over previous
import functools

import jax
import jax.numpy as jnp
from jax import lax
from jax.experimental import pallas as pl
from jax.experimental.pallas import tpu as pltpu
from jax.experimental.pallas import tpu_sc as plsc

N = 10000
NPAD = 10240
B = 8
K = 16
R = 256
NBLK = NPAD // R
BIG = 1e30
BIG2 = 2e30

SC_NC = 2
SC_NS = 16
SC_NW = SC_NC * SC_NS


def _dotH(a, b):
    return jnp.dot(a, b, precision=jax.lax.Precision.HIGHEST,
                   preferred_element_type=jnp.float32)


def _dotD(a, b):
    return jnp.dot(a, b, precision=jax.lax.Precision.DEFAULT,
                   preferred_element_type=jnp.float32)


def _elu(x):
    return jnp.where(x > 0, x, jnp.exp(jnp.minimum(x, 0.0)) - 1.0)


def _prep_body(y_ref, bc_ref, emb_ref, w1_ref, b1_ref, w2_ref, b2_ref,
               c_ref, cb_ref, st_ref):
    y = y_ref[...]
    oh = (y == lax.broadcasted_iota(jnp.int32, (1, 16), 1)).astype(jnp.float32)
    c = _elu(_dotH(oh, emb_ref[...]))
    c = _elu(_dotD(c, w1_ref[...]) + b1_ref[...])
    c = _dotD(c, w2_ref[...]) + b2_ref[...]
    c_ref[...] = c
    bc = bc_ref[...]
    m = (bc == lax.broadcasted_iota(jnp.int32, (1, B), 1)).astype(jnp.float32)
    cb_ref[...] = _dotH(m, c)
    lt = (bc < lax.broadcasted_iota(jnp.int32, (1, 16), 1)).astype(jnp.float32)
    st_ref[...] = jnp.sum(lt, axis=0, keepdims=True).astype(jnp.int32)


def _prep(y2, batch_col, emb_W, ce_W1, ce_b1, ce_W2, ce_b2):
    return pl.pallas_call(
        _prep_body,
        out_shape=(
            jax.ShapeDtypeStruct((B, 128), jnp.float32),
            jax.ShapeDtypeStruct((NPAD, 128), jnp.float32),
            jax.ShapeDtypeStruct((1, 16), jnp.int32),
        ),
    )(y2, batch_col, emb_W, ce_W1, ce_b1, ce_W2, ce_b2)


def _knn_body(x_ref, xt_ref, bc_ref, br_ref, idx_ref):
    blk = pl.program_id(0)
    xb = x_ref[...]
    xt = xt_ref[...]
    p = _dotD(xb, xt)
    sqj = jnp.sum(xt * xt, axis=0, keepdims=True)
    sqi = jnp.sum(xb * xb, axis=1, keepdims=True)
    score = (sqi - 2.0 * p) + sqj
    bc = bc_ref[...]
    br = br_ref[...]
    rowf = (blk * R + lax.broadcasted_iota(jnp.int32, (R, 1), 0)).astype(
        jnp.float32)
    colf = lax.broadcasted_iota(jnp.int32, (1, NPAD), 1).astype(jnp.float32)
    ok = (bc == br) & (bc < B) & (rowf != colf)
    score = jnp.where(ok, score, BIG)
    for t in range(K):
        m = jnp.min(score, axis=1, keepdims=True)
        cand = jnp.where(score == m, colf, float(NPAD))
        j = jnp.min(cand, axis=1, keepdims=True)
        j = jnp.minimum(j, float(NPAD - 1))
        idx_ref[:, t:t + 1] = j.astype(jnp.int32)
        score = jnp.where(colf == j, BIG2, score)


def _knn_full(xin, xint, batch_col, batch_row, d):
    return pl.pallas_call(
        _knn_body,
        grid=(NBLK,),
        in_specs=[
            pl.BlockSpec((R, d), lambda i: (i, 0)),
            pl.BlockSpec((d, NPAD), lambda i: (0, 0)),
            pl.BlockSpec((R, 1), lambda i: (i, 0)),
            pl.BlockSpec((1, NPAD), lambda i: (0, 0)),
        ],
        out_specs=pl.BlockSpec((R, K), lambda i: (i, 0)),
        out_shape=jax.ShapeDtypeStruct((NPAD, K), jnp.int32),
    )(xin, xint, batch_col, batch_row)


CC = 256
NWC = 16
NCH = NPAD // CC


def _knn_win_body(cl_ref, x_ref, xt_ref, bc_ref, br_ref, idx_ref, sc_ref):
    i = pl.program_id(0)
    j = pl.program_id(1)
    base = cl_ref[i]
    actual = base + j
    valid = actual < NCH
    xb = x_ref[...]
    bc = bc_ref[...]

    @pl.when(valid)
    def _():
        xt = xt_ref[...]
        p = _dotD(xb, xt)
        sqj = jnp.sum(xt * xt, axis=0, keepdims=True)
        sqi = jnp.sum(xb * xb, axis=1, keepdims=True)
        score = (sqi - 2.0 * p) + sqj
        br = br_ref[...]
        rowf = (i * R + lax.broadcasted_iota(jnp.int32, (R, 1), 0)).astype(
            jnp.float32)
        colf = (actual * CC + lax.broadcasted_iota(
            jnp.int32, (1, CC), 1)).astype(jnp.float32)
        ok = (bc == br) & (bc < B) & (rowf != colf)
        sc_ref[:, pl.ds(j * CC, CC)] = jnp.where(ok, score, BIG)

    @pl.when(jnp.logical_not(valid))
    def _():
        sc_ref[:, pl.ds(j * CC, CC)] = jnp.full((R, CC), BIG, jnp.float32)

    @pl.when(j == NWC - 1)
    def _():
        score = sc_ref[...]
        colf = (base * CC).astype(jnp.float32) + lax.broadcasted_iota(
            jnp.int32, (1, NWC * CC), 1).astype(jnp.float32)
        for t in range(K):
            m = jnp.min(score, axis=1, keepdims=True)
            cand = jnp.where(score == m, colf, float(NPAD))
            jv = jnp.min(cand, axis=1, keepdims=True)
            jv = jnp.minimum(jv, float(NPAD - 1))
            idx_ref[:, t:t + 1] = jv.astype(jnp.int32)
            score = jnp.where(colf == jv, BIG2, score)


def _knn_win(xin, xint, batch_col, batch_row, chunk_lo, d):
    grid_spec = pltpu.PrefetchScalarGridSpec(
        num_scalar_prefetch=1,
        grid=(NBLK, NWC),
        in_specs=[
            pl.BlockSpec((R, d), lambda i, j, cl: (i, 0)),
            pl.BlockSpec((d, CC),
                         lambda i, j, cl: (0, jnp.minimum(cl[i] + j, NCH - 1))),
            pl.BlockSpec((R, 1), lambda i, j, cl: (i, 0)),
            pl.BlockSpec((1, CC),
                         lambda i, j, cl: (0, jnp.minimum(cl[i] + j, NCH - 1))),
        ],
        out_specs=pl.BlockSpec((R, K), lambda i, j, cl: (i, 0)),
        scratch_shapes=[pltpu.VMEM((R, NWC * CC), jnp.float32)],
    )
    return pl.pallas_call(
        _knn_win_body,
        grid_spec=grid_spec,
        out_shape=jax.ShapeDtypeStruct((NPAD, K), jnp.int32),
    )(chunk_lo, xin, xint, batch_col, batch_row)


def _knn(xin, xint, batch_col, batch_row, batch_p, starts, d):
    rows = jnp.arange(NBLK) * R
    g_lo = jnp.minimum(batch_p[rows], B - 1)
    g_hi = jnp.minimum(batch_p[rows + R - 1], B - 1)
    lo_col = starts[g_lo]
    hi_col = jnp.maximum(starts[g_hi + 1], lo_col + 1)
    chunk_lo = (lo_col // CC).astype(jnp.int32)
    span = (hi_col - 1) // CC - chunk_lo + 1
    fits = jnp.max(span) <= NWC
    return lax.cond(
        fits,
        lambda: _knn_win(xin, xint, batch_col, batch_row, chunk_lo, d),
        lambda: _knn_full(xin, xint, batch_col, batch_row, d),
    )


def _sc_gather(table, idx_flat, h):
    nk = NPAD * K
    per_w = nk // SC_NW
    chunk = 64
    n_iter = per_w // chunk
    mesh = plsc.VectorSubcoreMesh(core_axis_name="c", subcore_axis_name="s")

    @functools.partial(
        pl.kernel,
        mesh=mesh,
        out_type=jax.ShapeDtypeStruct((nk, h), jnp.float32),
        scratch_types=[
            pltpu.VMEM((chunk,), jnp.int32),
            pltpu.VMEM((chunk, h), jnp.float32),
            pltpu.SemaphoreType.DMA,
        ],
    )
    def gk(table_hbm, idx_hbm, out_hbm, idx_v, rows_v, sem):
        wid = lax.axis_index("s") * SC_NC + lax.axis_index("c")
        base = wid * per_w

        def body(it, carry):
            off = base + it * chunk
            pltpu.sync_copy(idx_hbm.at[pl.ds(off, chunk)], idx_v)
            pltpu.async_copy(table_hbm.at[idx_v], rows_v, sem).wait()
            pltpu.sync_copy(rows_v, out_hbm.at[pl.ds(off, chunk)])
            return carry

        lax.fori_loop(0, n_iter, body, 0)

    return gk(table, idx_flat)


def _conv_body(x_ref, g_ref, bc_ref, w1_ref, b1_ref, w2_ref, b2_ref,
               o_ref, st_ref):
    xb = x_ref[...]
    w1 = w1_ref[...]
    w2 = w2_ref[...]
    b1 = b1_ref[...]
    acc = None
    for k in range(K):
        xj = g_ref[:, k, :]
        e = jnp.concatenate([xb, xj - xb], axis=1)
        h = _elu(_dotD(e, w1) + b1)
        p = _dotD(h, w2)
        acc = p if acc is None else jnp.maximum(acc, p)
    o = acc + b2_ref[...]
    o_ref[...] = o
    bc = bc_ref[...]
    m8 = (bc == lax.broadcasted_iota(jnp.int32, (1, B), 1)).astype(jnp.float32)
    cnt = jnp.sum(m8, axis=0, keepdims=True)
    s1 = jnp.sum(m8 * jnp.sum(o, axis=1, keepdims=True), axis=0, keepdims=True)
    part = jnp.concatenate([cnt, s1, jnp.zeros((6, B), jnp.float32)], axis=0)

    @pl.when(pl.program_id(0) == 0)
    def _():
        st_ref[...] = part

    @pl.when(pl.program_id(0) != 0)
    def _():
        st_ref[...] += part


def _conv(xin, g3, batch_col, w1, b1, w2, b2, d, h, o):
    return pl.pallas_call(
        _conv_body,
        grid=(NBLK,),
        in_specs=[
            pl.BlockSpec((R, d), lambda i: (i, 0)),
            pl.BlockSpec((R, K, d), lambda i: (i, 0, 0)),
            pl.BlockSpec((R, 1), lambda i: (i, 0)),
            pl.BlockSpec((2 * d, h), lambda i: (0, 0)),
            pl.BlockSpec((1, h), lambda i: (0, 0)),
            pl.BlockSpec((h, o), lambda i: (0, 0)),
            pl.BlockSpec((1, o), lambda i: (0, 0)),
        ],
        out_specs=(
            pl.BlockSpec((R, o), lambda i: (i, 0)),
            pl.BlockSpec((8, B), lambda i: (0, 0)),
        ),
        out_shape=(
            jax.ShapeDtypeStruct((NPAD, o), jnp.float32),
            jax.ShapeDtypeStruct((8, B), jnp.float32),
        ),
    )(xin, g3, batch_col, w1, b1, w2, b2)


def _stats2_body(x_ref, bc_ref, st1_ref, st2_ref):
    x = x_ref[...]
    f = x.shape[1]
    bc = bc_ref[...]
    cnt = st1_ref[0:1, :]
    s1 = st1_ref[1:2, :]
    denom = jnp.maximum(cnt, 1.0) * float(f)
    mean = s1 / denom
    m8 = (bc == lax.broadcasted_iota(jnp.int32, (1, B), 1)).astype(jnp.float32)
    mean_b = jnp.sum(m8 * mean, axis=1, keepdims=True)
    xc = x - mean_b
    s2 = jnp.sum(m8 * jnp.sum(xc * xc, axis=1, keepdims=True), axis=0,
                 keepdims=True)
    part = jnp.concatenate([s2, jnp.zeros((7, B), jnp.float32)], axis=0)

    @pl.when(pl.program_id(0) == 0)
    def _():
        st2_ref[...] = part

    @pl.when(pl.program_id(0) != 0)
    def _():
        st2_ref[...] += part


def _stats2(x, batch_col, st1, f):
    return pl.pallas_call(
        _stats2_body,
        grid=(NBLK,),
        in_specs=[
            pl.BlockSpec((R, f), lambda i: (i, 0)),
            pl.BlockSpec((R, 1), lambda i: (i, 0)),
            pl.BlockSpec((8, B), lambda i: (0, 0)),
        ],
        out_specs=pl.BlockSpec((8, B), lambda i: (0, 0)),
        out_shape=jax.ShapeDtypeStruct((8, B), jnp.float32),
    )(x, batch_col, st1)


def _ln_body(x_ref, bc_ref, st1_ref, st2_ref, w_ref, b_ref, o_ref):
    x = x_ref[...]
    f = x.shape[1]
    bc = bc_ref[...]
    cnt = st1_ref[0:1, :]
    s1 = st1_ref[1:2, :]
    s2 = st2_ref[0:1, :]
    denom = jnp.maximum(cnt, 1.0) * float(f)
    mean = s1 / denom
    var = s2 / denom
    m8 = (bc == lax.broadcasted_iota(jnp.int32, (1, B), 1)).astype(jnp.float32)
    mean_b = jnp.sum(m8 * mean, axis=1, keepdims=True)
    var_b = jnp.sum(m8 * var, axis=1, keepdims=True)
    out = (x - mean_b) * lax.rsqrt(var_b + 1e-5) * w_ref[...] + b_ref[...]
    out = _elu(out)
    o_ref[...] = jnp.where(bc < B, out, 0.0)


def _graph_ln(x, batch_col, st1, st2, w, b, f):
    return pl.pallas_call(
        _ln_body,
        grid=(NBLK,),
        in_specs=[
            pl.BlockSpec((R, f), lambda i: (i, 0)),
            pl.BlockSpec((R, 1), lambda i: (i, 0)),
            pl.BlockSpec((8, B), lambda i: (0, 0)),
            pl.BlockSpec((8, B), lambda i: (0, 0)),
            pl.BlockSpec((1, f), lambda i: (0, 0)),
            pl.BlockSpec((1, f), lambda i: (0, 0)),
        ],
        out_specs=pl.BlockSpec((R, f), lambda i: (i, 0)),
        out_shape=jax.ShapeDtypeStruct((NPAD, f), jnp.float32),
    )(x, batch_col, st1, st2, w, b)


def _head_body(x_ref, br_ref, c_ref, w1a_ref, w1b_ref, b1_ref, nw1_ref,
               nb1_ref, w2_ref, b2_ref, nw2_ref, nb2_ref, w3_ref, b3_ref,
               o_ref):
    br = br_ref[...]
    m8t = (lax.broadcasted_iota(jnp.int32, (B, 1), 0) == br).astype(
        jnp.float32)
    cnt = jnp.sum(m8t, axis=1, keepdims=True)
    pooled = _dotH(m8t, x_ref[...]) / jnp.maximum(cnt, 1.0)

    def ln(v, w, bb):
        mu = jnp.mean(v, axis=-1, keepdims=True)
        va = jnp.mean((v - mu) ** 2, axis=-1, keepdims=True)
        return (v - mu) * lax.rsqrt(va + 1e-5) * w + bb

    h = (_dotD(pooled, w1a_ref[...]) + _dotD(c_ref[...], w1b_ref[...])
         + b1_ref[...])
    h = _elu(ln(h, nw1_ref[...], nb1_ref[...]))
    h = _dotD(h, w2_ref[...]) + b2_ref[...]
    h = _elu(ln(h, nw2_ref[...], nb2_ref[...]))
    o_ref[...] = _dotD(h, w3_ref[...]) + b3_ref[...]


def _head(x, batch_row, c, fw1a, fw1b, fb1, fnw1, fnb1, fw2, fb2, fnw2,
          fnb2, fw3, fb3):
    return pl.pallas_call(
        _head_body,
        out_shape=jax.ShapeDtypeStruct((B, 1), jnp.float32),
    )(x, batch_row, c, fw1a, fw1b, fb1, fnw1, fnb1, fw2, fb2, fnw2, fnb2,
      fw3, fb3)


def kernel(pos, y, batch, emb_W, ce_W1, ce_b1, ce_W2, ce_b2,
           cw1_0, cb1_0, cw2_0, cb2_0, nw_0, nb_0,
           cw1_1, cb1_1, cw2_1, cb2_1, nw_1, nb_1,
           cw1_2, cb1_2, cw2_2, cb2_2, nw_2, nb_2,
           fw1, fb1, fnw1, fnb1, fw2, fb2, fnw2, fnb2, fw3, fb3):
    batch = batch.astype(jnp.int32)
    batch_p = jnp.pad(batch, (0, NPAD - N), constant_values=B)
    batch_col = batch_p.reshape(NPAD, 1)
    batch_row = batch_p.reshape(1, NPAD)
    y2 = y.astype(jnp.int32).reshape(B, 1)

    c, cb, starts2d = _prep(y2, batch_col, emb_W, ce_W1, ce_b1, ce_W2, ce_b2)
    starts = starts2d.reshape(16)

    hc = [3, 128, 256, 512]
    conv_params = [(cw1_0, cb1_0, cw2_0, cb2_0),
                   (cw1_1, cb1_1, cw2_1, cb2_1),
                   (cw1_2, cb1_2, cw2_2, cb2_2)]
    norm_params = [(nw_0, nb_0), (nw_1, nb_1), (nw_2, nb_2)]

    x = jnp.pad(pos, ((0, NPAD - N), (0, 0)))
    for i in range(3):
        in_c = hc[i] + 128
        hid = hc[i] * 2
        out = hc[i + 1]
        w1, b1, w2, b2 = conv_params[i]
        nw, nb = norm_params[i]
        dp = max(256, ((in_c + 127) // 128) * 128)
        hp = max(128, ((hid + 127) // 128) * 128)

        xin = jnp.concatenate([x, cb], axis=1)
        if xin.shape[1] < dp:
            xin = jnp.pad(xin, ((0, 0), (0, dp - xin.shape[1])))
        w1a = jnp.pad(w1[:in_c], ((0, dp - in_c), (0, 0)))
        w1b = jnp.pad(w1[in_c:], ((0, dp - in_c), (0, 0)))
        w1p = jnp.concatenate([w1a, w1b], axis=0)
        if hid < hp:
            w1p = jnp.pad(w1p, ((0, 0), (0, hp - hid)))
            b1p = jnp.pad(b1, (0, hp - hid))
            w2p = jnp.pad(w2, ((0, hp - hid), (0, 0)))
        else:
            b1p = b1
            w2p = w2

        idx = _knn(xin, xin.T, batch_col, batch_row, batch_p, starts, dp)
        g = _sc_gather(xin, idx.reshape(NPAD * K), dp)
        xconv, st1 = _conv(xin, g.reshape(NPAD, K, dp), batch_col, w1p,
                           b1p.reshape(1, hp), w2p, b2.reshape(1, out),
                           dp, hp, out)
        st2 = _stats2(xconv, batch_col, st1, out)
        x = _graph_ln(xconv, batch_col, st1, st2, nw.reshape(1, out),
                      nb.reshape(1, out), out)

    fw1a = fw1[:hc[-1]]
    fw1b = fw1[hc[-1]:]
    return _head(x, batch_row, c, fw1a, fw1b, fb1.reshape(1, -1),
                 fnw1.reshape(1, -1), fnb1.reshape(1, -1), fw2,
                 fb2.reshape(1, -1), fnw2.reshape(1, -1), fnb2.reshape(1, -1),
                 fw3, fb3.reshape(1, -1))


# speedup vs baseline: 9.2518x; 1.0561x over previous
import functools

import jax
import jax.numpy as jnp
from jax import lax
from jax.experimental import pallas as pl
from jax.experimental.pallas import tpu as pltpu
from jax.experimental.pallas import tpu_sc as plsc

N = 10000
NPAD = 10240
B = 8
K = 16
R = 256
NBLK = NPAD // R
BIG = 1e30
BIG2 = 2e30

SC_NC = 2
SC_NS = 16
SC_NW = SC_NC * SC_NS


def _dotH(a, b):
    return jnp.dot(a, b, precision=jax.lax.Precision.HIGHEST,
                   preferred_element_type=jnp.float32)


def _dotD(a, b):
    return jnp.dot(a, b, precision=jax.lax.Precision.DEFAULT,
                   preferred_element_type=jnp.float32)


def _elu(x):
    return jnp.where(x > 0, x, jnp.exp(jnp.minimum(x, 0.0)) - 1.0)


def _prep_body(y_ref, bc_ref, emb_ref, w1_ref, b1_ref, w2_ref, b2_ref,
               c_ref, cb_ref, st_ref):
    y = y_ref[...]
    oh = (y == lax.broadcasted_iota(jnp.int32, (1, 16), 1)).astype(jnp.float32)
    c = _elu(_dotH(oh, emb_ref[...]))
    c = _elu(_dotD(c, w1_ref[...]) + b1_ref[...])
    c = _dotD(c, w2_ref[...]) + b2_ref[...]
    c_ref[...] = c
    bc = bc_ref[...]
    m = (bc == lax.broadcasted_iota(jnp.int32, (1, B), 1)).astype(jnp.float32)
    cb_ref[...] = _dotH(m, c)
    lt = (bc < lax.broadcasted_iota(jnp.int32, (1, 16), 1)).astype(jnp.float32)
    st_ref[...] = jnp.sum(lt, axis=0, keepdims=True).astype(jnp.int32)


def _prep(y2, batch_col, emb_W, ce_W1, ce_b1, ce_W2, ce_b2):
    return pl.pallas_call(
        _prep_body,
        out_shape=(
            jax.ShapeDtypeStruct((B, 128), jnp.float32),
            jax.ShapeDtypeStruct((NPAD, 128), jnp.float32),
            jax.ShapeDtypeStruct((1, 16), jnp.int32),
        ),
    )(y2, batch_col, emb_W, ce_W1, ce_b1, ce_W2, ce_b2)


def _knn_body(x_ref, xt_ref, bc_ref, br_ref, idx_ref):
    blk = pl.program_id(0)
    xb = x_ref[...]
    xt = xt_ref[...]
    p = _dotD(xb, xt)
    sqj = jnp.sum(xt * xt, axis=0, keepdims=True)
    sqi = jnp.sum(xb * xb, axis=1, keepdims=True)
    score = (sqi - 2.0 * p) + sqj
    bc = bc_ref[...]
    br = br_ref[...]
    rowf = (blk * R + lax.broadcasted_iota(jnp.int32, (R, 1), 0)).astype(
        jnp.float32)
    colf = lax.broadcasted_iota(jnp.int32, (1, NPAD), 1).astype(jnp.float32)
    ok = (bc == br) & (bc < B) & (rowf != colf)
    score = jnp.where(ok, score, BIG)
    for t in range(K):
        m = jnp.min(score, axis=1, keepdims=True)
        cand = jnp.where(score == m, colf, float(NPAD))
        j = jnp.min(cand, axis=1, keepdims=True)
        j = jnp.minimum(j, float(NPAD - 1))
        idx_ref[:, t:t + 1] = j.astype(jnp.int32)
        score = jnp.where(colf == j, BIG2, score)


def _knn_full(xin, xint, batch_col, batch_row, d):
    return pl.pallas_call(
        _knn_body,
        grid=(NBLK,),
        in_specs=[
            pl.BlockSpec((R, d), lambda i: (i, 0)),
            pl.BlockSpec((d, NPAD), lambda i: (0, 0)),
            pl.BlockSpec((R, 1), lambda i: (i, 0)),
            pl.BlockSpec((1, NPAD), lambda i: (0, 0)),
        ],
        out_specs=pl.BlockSpec((R, K), lambda i: (i, 0)),
        out_shape=jax.ShapeDtypeStruct((NPAD, K), jnp.int32),
    )(xin, xint, batch_col, batch_row)


CC = 256
NWC = 16
NCH = NPAD // CC


def _knn_win_body(cl_ref, x_ref, xt_ref, bc_ref, br_ref, idx_ref, sc_ref):
    i = pl.program_id(0)
    j = pl.program_id(1)
    base = cl_ref[i]
    actual = base + j
    valid = actual < NCH
    xb = x_ref[...]
    bc = bc_ref[...]

    @pl.when(valid)
    def _():
        xt = xt_ref[...]
        p = _dotD(xb, xt)
        sqj = jnp.sum(xt * xt, axis=0, keepdims=True)
        sqi = jnp.sum(xb * xb, axis=1, keepdims=True)
        score = (sqi - 2.0 * p) + sqj
        br = br_ref[...]
        rowf = (i * R + lax.broadcasted_iota(jnp.int32, (R, 1), 0)).astype(
            jnp.float32)
        colf = (actual * CC + lax.broadcasted_iota(
            jnp.int32, (1, CC), 1)).astype(jnp.float32)
        ok = (bc == br) & (bc < B) & (rowf != colf)
        sc_ref[:, pl.ds(j * CC, CC)] = jnp.where(ok, score, BIG)

    @pl.when(jnp.logical_not(valid))
    def _():
        sc_ref[:, pl.ds(j * CC, CC)] = jnp.full((R, CC), BIG, jnp.float32)

    @pl.when(j == NWC - 1)
    def _():
        score = sc_ref[...]
        colf = (base * CC).astype(jnp.float32) + lax.broadcasted_iota(
            jnp.int32, (1, NWC * CC), 1).astype(jnp.float32)
        for t in range(K):
            m = jnp.min(score, axis=1, keepdims=True)
            cand = jnp.where(score == m, colf, float(NPAD))
            jv = jnp.min(cand, axis=1, keepdims=True)
            jv = jnp.minimum(jv, float(NPAD - 1))
            idx_ref[:, t:t + 1] = jv.astype(jnp.int32)
            score = jnp.where(colf == jv, BIG2, score)


def _knn_win(xin, xint, batch_col, batch_row, chunk_lo, d):
    grid_spec = pltpu.PrefetchScalarGridSpec(
        num_scalar_prefetch=1,
        grid=(NBLK, NWC),
        in_specs=[
            pl.BlockSpec((R, d), lambda i, j, cl: (i, 0)),
            pl.BlockSpec((d, CC),
                         lambda i, j, cl: (0, jnp.minimum(cl[i] + j, NCH - 1))),
            pl.BlockSpec((R, 1), lambda i, j, cl: (i, 0)),
            pl.BlockSpec((1, CC),
                         lambda i, j, cl: (0, jnp.minimum(cl[i] + j, NCH - 1))),
        ],
        out_specs=pl.BlockSpec((R, K), lambda i, j, cl: (i, 0)),
        scratch_shapes=[pltpu.VMEM((R, NWC * CC), jnp.float32)],
    )
    return pl.pallas_call(
        _knn_win_body,
        grid_spec=grid_spec,
        out_shape=jax.ShapeDtypeStruct((NPAD, K), jnp.int32),
    )(chunk_lo, xin, xint, batch_col, batch_row)


def _knn(xin, xint, batch_col, batch_row, batch_p, starts, d):
    rows = jnp.arange(NBLK) * R
    g_lo = jnp.minimum(batch_p[rows], B - 1)
    g_hi = jnp.minimum(batch_p[rows + R - 1], B - 1)
    lo_col = starts[g_lo]
    hi_col = jnp.maximum(starts[g_hi + 1], lo_col + 1)
    chunk_lo = (lo_col // CC).astype(jnp.int32)
    span = (hi_col - 1) // CC - chunk_lo + 1
    fits = jnp.max(span) <= NWC
    return lax.cond(
        fits,
        lambda: _knn_win(xin, xint, batch_col, batch_row, chunk_lo, d),
        lambda: _knn_full(xin, xint, batch_col, batch_row, d),
    )


def _sc_gather(table, idx_flat, h):
    nk = NPAD * K
    per_w = nk // SC_NW
    chunk = 64
    n_iter = per_w // chunk
    mesh = plsc.VectorSubcoreMesh(core_axis_name="c", subcore_axis_name="s")

    n2 = n_iter // 2

    @functools.partial(
        pl.kernel,
        mesh=mesh,
        out_type=jax.ShapeDtypeStruct((nk, h), jnp.float32),
        scratch_types=[
            pltpu.VMEM((chunk,), jnp.int32),
            pltpu.VMEM((chunk,), jnp.int32),
            pltpu.VMEM((chunk, h), jnp.float32),
            pltpu.VMEM((chunk, h), jnp.float32),
            pltpu.SemaphoreType.DMA,
            pltpu.SemaphoreType.DMA,
        ],
    )
    def gk(table_hbm, idx_hbm, out_hbm, idx_a, idx_b, rows_a, rows_b,
           sem_a, sem_b):
        wid = lax.axis_index("s") * SC_NC + lax.axis_index("c")
        base = wid * per_w

        pltpu.sync_copy(idx_hbm.at[pl.ds(base, chunk)], idx_a)
        pltpu.async_copy(table_hbm.at[idx_a], rows_a, sem_a)

        def body(t, carry):
            off0 = base + (2 * t) * chunk
            off1 = off0 + chunk
            pltpu.sync_copy(idx_hbm.at[pl.ds(off1, chunk)], idx_b)
            pltpu.async_copy(table_hbm.at[idx_b], rows_b, sem_b)
            pltpu.make_async_copy(table_hbm.at[idx_a], rows_a, sem_a).wait()
            pltpu.sync_copy(rows_a, out_hbm.at[pl.ds(off0, chunk)])

            @pl.when(t < n2 - 1)
            def _():
                pltpu.sync_copy(idx_hbm.at[pl.ds(off1 + chunk, chunk)], idx_a)
                pltpu.async_copy(table_hbm.at[idx_a], rows_a, sem_a)

            pltpu.make_async_copy(table_hbm.at[idx_b], rows_b, sem_b).wait()
            pltpu.sync_copy(rows_b, out_hbm.at[pl.ds(off1, chunk)])
            return carry

        lax.fori_loop(0, n2, body, 0)

    return gk(table, idx_flat)


def _conv_body(x_ref, g_ref, bc_ref, w1_ref, b1_ref, w2_ref, b2_ref,
               o_ref, st_ref):
    xb = x_ref[...]
    w1 = w1_ref[...]
    w2 = w2_ref[...]
    b1 = b1_ref[...]
    acc = None
    for k in range(K):
        xj = g_ref[:, k, :]
        e = jnp.concatenate([xb, xj - xb], axis=1)
        h = _elu(_dotD(e, w1) + b1)
        p = _dotD(h, w2)
        acc = p if acc is None else jnp.maximum(acc, p)
    o = acc + b2_ref[...]
    o_ref[...] = o
    bc = bc_ref[...]
    m8 = (bc == lax.broadcasted_iota(jnp.int32, (1, B), 1)).astype(jnp.float32)
    cnt = jnp.sum(m8, axis=0, keepdims=True)
    s1 = jnp.sum(m8 * jnp.sum(o, axis=1, keepdims=True), axis=0, keepdims=True)
    part = jnp.concatenate([cnt, s1, jnp.zeros((6, B), jnp.float32)], axis=0)

    @pl.when(pl.program_id(0) == 0)
    def _():
        st_ref[...] = part

    @pl.when(pl.program_id(0) != 0)
    def _():
        st_ref[...] += part


def _conv(xin, g3, batch_col, w1, b1, w2, b2, d, h, o):
    return pl.pallas_call(
        _conv_body,
        grid=(NBLK,),
        in_specs=[
            pl.BlockSpec((R, d), lambda i: (i, 0)),
            pl.BlockSpec((R, K, d), lambda i: (i, 0, 0)),
            pl.BlockSpec((R, 1), lambda i: (i, 0)),
            pl.BlockSpec((2 * d, h), lambda i: (0, 0)),
            pl.BlockSpec((1, h), lambda i: (0, 0)),
            pl.BlockSpec((h, o), lambda i: (0, 0)),
            pl.BlockSpec((1, o), lambda i: (0, 0)),
        ],
        out_specs=(
            pl.BlockSpec((R, o), lambda i: (i, 0)),
            pl.BlockSpec((8, B), lambda i: (0, 0)),
        ),
        out_shape=(
            jax.ShapeDtypeStruct((NPAD, o), jnp.float32),
            jax.ShapeDtypeStruct((8, B), jnp.float32),
        ),
    )(xin, g3, batch_col, w1, b1, w2, b2)


def _stats2_body(x_ref, bc_ref, st1_ref, st2_ref):
    x = x_ref[...]
    f = x.shape[1]
    bc = bc_ref[...]
    cnt = st1_ref[0:1, :]
    s1 = st1_ref[1:2, :]
    denom = jnp.maximum(cnt, 1.0) * float(f)
    mean = s1 / denom
    m8 = (bc == lax.broadcasted_iota(jnp.int32, (1, B), 1)).astype(jnp.float32)
    mean_b = jnp.sum(m8 * mean, axis=1, keepdims=True)
    xc = x - mean_b
    s2 = jnp.sum(m8 * jnp.sum(xc * xc, axis=1, keepdims=True), axis=0,
                 keepdims=True)
    part = jnp.concatenate([s2, jnp.zeros((7, B), jnp.float32)], axis=0)

    @pl.when(pl.program_id(0) == 0)
    def _():
        st2_ref[...] = part

    @pl.when(pl.program_id(0) != 0)
    def _():
        st2_ref[...] += part


def _stats2(x, batch_col, st1, f):
    return pl.pallas_call(
        _stats2_body,
        grid=(NBLK,),
        in_specs=[
            pl.BlockSpec((R, f), lambda i: (i, 0)),
            pl.BlockSpec((R, 1), lambda i: (i, 0)),
            pl.BlockSpec((8, B), lambda i: (0, 0)),
        ],
        out_specs=pl.BlockSpec((8, B), lambda i: (0, 0)),
        out_shape=jax.ShapeDtypeStruct((8, B), jnp.float32),
    )(x, batch_col, st1)


def _ln_body(x_ref, bc_ref, st1_ref, st2_ref, w_ref, b_ref, o_ref):
    x = x_ref[...]
    f = x.shape[1]
    bc = bc_ref[...]
    cnt = st1_ref[0:1, :]
    s1 = st1_ref[1:2, :]
    s2 = st2_ref[0:1, :]
    denom = jnp.maximum(cnt, 1.0) * float(f)
    mean = s1 / denom
    var = s2 / denom
    m8 = (bc == lax.broadcasted_iota(jnp.int32, (1, B), 1)).astype(jnp.float32)
    mean_b = jnp.sum(m8 * mean, axis=1, keepdims=True)
    var_b = jnp.sum(m8 * var, axis=1, keepdims=True)
    out = (x - mean_b) * lax.rsqrt(var_b + 1e-5) * w_ref[...] + b_ref[...]
    out = _elu(out)
    o_ref[...] = jnp.where(bc < B, out, 0.0)


def _graph_ln(x, batch_col, st1, st2, w, b, f):
    return pl.pallas_call(
        _ln_body,
        grid=(NBLK,),
        in_specs=[
            pl.BlockSpec((R, f), lambda i: (i, 0)),
            pl.BlockSpec((R, 1), lambda i: (i, 0)),
            pl.BlockSpec((8, B), lambda i: (0, 0)),
            pl.BlockSpec((8, B), lambda i: (0, 0)),
            pl.BlockSpec((1, f), lambda i: (0, 0)),
            pl.BlockSpec((1, f), lambda i: (0, 0)),
        ],
        out_specs=pl.BlockSpec((R, f), lambda i: (i, 0)),
        out_shape=jax.ShapeDtypeStruct((NPAD, f), jnp.float32),
    )(x, batch_col, st1, st2, w, b)


def _head_body(x_ref, br_ref, c_ref, w1a_ref, w1b_ref, b1_ref, nw1_ref,
               nb1_ref, w2_ref, b2_ref, nw2_ref, nb2_ref, w3_ref, b3_ref,
               o_ref):
    br = br_ref[...]
    m8t = (lax.broadcasted_iota(jnp.int32, (B, 1), 0) == br).astype(
        jnp.float32)
    cnt = jnp.sum(m8t, axis=1, keepdims=True)
    pooled = _dotH(m8t, x_ref[...]) / jnp.maximum(cnt, 1.0)

    def ln(v, w, bb):
        mu = jnp.mean(v, axis=-1, keepdims=True)
        va = jnp.mean((v - mu) ** 2, axis=-1, keepdims=True)
        return (v - mu) * lax.rsqrt(va + 1e-5) * w + bb

    h = (_dotD(pooled, w1a_ref[...]) + _dotD(c_ref[...], w1b_ref[...])
         + b1_ref[...])
    h = _elu(ln(h, nw1_ref[...], nb1_ref[...]))
    h = _dotD(h, w2_ref[...]) + b2_ref[...]
    h = _elu(ln(h, nw2_ref[...], nb2_ref[...]))
    o_ref[...] = _dotD(h, w3_ref[...]) + b3_ref[...]


def _head(x, batch_row, c, fw1a, fw1b, fb1, fnw1, fnb1, fw2, fb2, fnw2,
          fnb2, fw3, fb3):
    return pl.pallas_call(
        _head_body,
        out_shape=jax.ShapeDtypeStruct((B, 1), jnp.float32),
    )(x, batch_row, c, fw1a, fw1b, fb1, fnw1, fnb1, fw2, fb2, fnw2, fnb2,
      fw3, fb3)


def kernel(pos, y, batch, emb_W, ce_W1, ce_b1, ce_W2, ce_b2,
           cw1_0, cb1_0, cw2_0, cb2_0, nw_0, nb_0,
           cw1_1, cb1_1, cw2_1, cb2_1, nw_1, nb_1,
           cw1_2, cb1_2, cw2_2, cb2_2, nw_2, nb_2,
           fw1, fb1, fnw1, fnb1, fw2, fb2, fnw2, fnb2, fw3, fb3):
    batch = batch.astype(jnp.int32)
    batch_p = jnp.pad(batch, (0, NPAD - N), constant_values=B)
    batch_col = batch_p.reshape(NPAD, 1)
    batch_row = batch_p.reshape(1, NPAD)
    y2 = y.astype(jnp.int32).reshape(B, 1)

    c, cb, starts2d = _prep(y2, batch_col, emb_W, ce_W1, ce_b1, ce_W2, ce_b2)
    starts = starts2d.reshape(16)

    hc = [3, 128, 256, 512]
    conv_params = [(cw1_0, cb1_0, cw2_0, cb2_0),
                   (cw1_1, cb1_1, cw2_1, cb2_1),
                   (cw1_2, cb1_2, cw2_2, cb2_2)]
    norm_params = [(nw_0, nb_0), (nw_1, nb_1), (nw_2, nb_2)]

    x = jnp.pad(pos, ((0, NPAD - N), (0, 0)))
    for i in range(3):
        in_c = hc[i] + 128
        hid = hc[i] * 2
        out = hc[i + 1]
        w1, b1, w2, b2 = conv_params[i]
        nw, nb = norm_params[i]
        dp = max(256, ((in_c + 127) // 128) * 128)
        hp = max(128, ((hid + 127) // 128) * 128)

        xin = jnp.concatenate([x, cb], axis=1)
        if xin.shape[1] < dp:
            xin = jnp.pad(xin, ((0, 0), (0, dp - xin.shape[1])))
        w1a = jnp.pad(w1[:in_c], ((0, dp - in_c), (0, 0)))
        w1b = jnp.pad(w1[in_c:], ((0, dp - in_c), (0, 0)))
        w1p = jnp.concatenate([w1a, w1b], axis=0)
        if hid < hp:
            w1p = jnp.pad(w1p, ((0, 0), (0, hp - hid)))
            b1p = jnp.pad(b1, (0, hp - hid))
            w2p = jnp.pad(w2, ((0, hp - hid), (0, 0)))
        else:
            b1p = b1
            w2p = w2

        idx = _knn(xin, xin.T, batch_col, batch_row, batch_p, starts, dp)
        g = _sc_gather(xin, idx.reshape(NPAD * K), dp)
        xconv, st1 = _conv(xin, g.reshape(NPAD, K, dp), batch_col, w1p,
                           b1p.reshape(1, hp), w2p, b2.reshape(1, out),
                           dp, hp, out)
        st2 = _stats2(xconv, batch_col, st1, out)
        x = _graph_ln(xconv, batch_col, st1, st2, nw.reshape(1, out),
                      nb.reshape(1, out), out)

    fw1a = fw1[:hc[-1]]
    fw1b = fw1[hc[-1]:]
    return _head(x, batch_row, c, fw1a, fw1b, fb1.reshape(1, -1),
                 fnw1.reshape(1, -1), fnb1.reshape(1, -1), fw2,
                 fb2.reshape(1, -1), fnw2.reshape(1, -1), fnb2.reshape(1, -1),
                 fw3, fb3.reshape(1, -1))


# speedup vs baseline: 10.8068x; 1.1681x over previous
import functools

import jax
import jax.numpy as jnp
from jax import lax
from jax.experimental import pallas as pl
from jax.experimental.pallas import tpu as pltpu
from jax.experimental.pallas import tpu_sc as plsc

N = 10000
NPAD = 10240
B = 8
K = 16
R = 256
NBLK = NPAD // R
BIG = 1e30
BIG2 = 2e30

SC_NC = 2
SC_NS = 16
SC_NW = SC_NC * SC_NS


def _dotH(a, b):
    return jnp.dot(a, b, precision=jax.lax.Precision.HIGHEST,
                   preferred_element_type=jnp.float32)


def _dotD(a, b):
    return jnp.dot(a, b, precision=jax.lax.Precision.DEFAULT,
                   preferred_element_type=jnp.float32)


def _elu(x):
    return jnp.where(x > 0, x, jnp.exp(jnp.minimum(x, 0.0)) - 1.0)


def _prep_body(y_ref, bc_ref, emb_ref, w1_ref, b1_ref, w2_ref, b2_ref,
               c_ref, cb_ref, st_ref):
    y = y_ref[...]
    oh = (y == lax.broadcasted_iota(jnp.int32, (1, 16), 1)).astype(jnp.float32)
    c = _elu(_dotH(oh, emb_ref[...]))
    c = _elu(_dotD(c, w1_ref[...]) + b1_ref[...])
    c = _dotD(c, w2_ref[...]) + b2_ref[...]
    c_ref[...] = c
    bc = bc_ref[...]
    m = (bc == lax.broadcasted_iota(jnp.int32, (1, B), 1)).astype(jnp.float32)
    cb_ref[...] = _dotH(m, c)
    lt = (bc < lax.broadcasted_iota(jnp.int32, (1, 16), 1)).astype(jnp.float32)
    st_ref[...] = jnp.sum(lt, axis=0, keepdims=True).astype(jnp.int32)


def _prep(y2, batch_col, emb_W, ce_W1, ce_b1, ce_W2, ce_b2):
    return pl.pallas_call(
        _prep_body,
        out_shape=(
            jax.ShapeDtypeStruct((B, 128), jnp.float32),
            jax.ShapeDtypeStruct((NPAD, 128), jnp.float32),
            jax.ShapeDtypeStruct((1, 16), jnp.int32),
        ),
    )(y2, batch_col, emb_W, ce_W1, ce_b1, ce_W2, ce_b2)


def _knn_body(x_ref, xt_ref, bc_ref, br_ref, idx_ref):
    blk = pl.program_id(0)
    xb = x_ref[...]
    xt = xt_ref[...]
    p = _dotD(xb, xt)
    sqj = jnp.sum(xt * xt, axis=0, keepdims=True)
    sqi = jnp.sum(xb * xb, axis=1, keepdims=True)
    score = (sqi - 2.0 * p) + sqj
    bc = bc_ref[...]
    br = br_ref[...]
    rowf = (blk * R + lax.broadcasted_iota(jnp.int32, (R, 1), 0)).astype(
        jnp.float32)
    colf = lax.broadcasted_iota(jnp.int32, (1, NPAD), 1).astype(jnp.float32)
    ok = (bc == br) & (bc < B) & (rowf != colf)
    score = jnp.where(ok, score, BIG)
    for t in range(K):
        m = jnp.min(score, axis=1, keepdims=True)
        cand = jnp.where(score == m, colf, float(NPAD))
        j = jnp.min(cand, axis=1, keepdims=True)
        j = jnp.minimum(j, float(NPAD - 1))
        idx_ref[:, t:t + 1] = j.astype(jnp.int32)
        score = jnp.where(colf == j, BIG2, score)


def _knn_full(xin, xint, batch_col, batch_row, d):
    return pl.pallas_call(
        _knn_body,
        grid=(NBLK,),
        in_specs=[
            pl.BlockSpec((R, d), lambda i: (i, 0)),
            pl.BlockSpec((d, NPAD), lambda i: (0, 0)),
            pl.BlockSpec((R, 1), lambda i: (i, 0)),
            pl.BlockSpec((1, NPAD), lambda i: (0, 0)),
        ],
        out_specs=pl.BlockSpec((R, K), lambda i: (i, 0)),
        out_shape=jax.ShapeDtypeStruct((NPAD, K), jnp.int32),
    )(xin, xint, batch_col, batch_row)


CC = 256
NWC = 16
NCH = NPAD // CC


def _knn_win_body(cl_ref, sp_ref, x_ref, xt_ref, bc_ref, br_ref, idx_ref,
                  sc_ref):
    i = pl.program_id(0)
    j = pl.program_id(1)
    base = cl_ref[i]
    actual = base + j
    valid = actual < NCH
    xb = x_ref[...]
    bc = bc_ref[...]

    @pl.when(valid)
    def _():
        xt = xt_ref[...]
        p = _dotD(xb, xt)
        sqj = jnp.sum(xt * xt, axis=0, keepdims=True)
        sqi = jnp.sum(xb * xb, axis=1, keepdims=True)
        score = (sqi - 2.0 * p) + sqj
        br = br_ref[...]
        rowf = (i * R + lax.broadcasted_iota(jnp.int32, (R, 1), 0)).astype(
            jnp.float32)
        colf = (actual * CC + lax.broadcasted_iota(
            jnp.int32, (1, CC), 1)).astype(jnp.float32)
        ok = (bc == br) & (bc < B) & (rowf != colf)
        sc_ref[:, pl.ds(j * CC, CC)] = jnp.where(ok, score, BIG)

    @pl.when(jnp.logical_not(valid))
    def _():
        sc_ref[:, pl.ds(j * CC, CC)] = jnp.full((R, CC), BIG, jnp.float32)

    def extract(w):
        score = sc_ref[:, :w]
        colf = (base * CC).astype(jnp.float32) + lax.broadcasted_iota(
            jnp.int32, (1, w), 1).astype(jnp.float32)
        for t in range(K):
            m = jnp.min(score, axis=1, keepdims=True)
            cand = jnp.where(score == m, colf, float(NPAD))
            jv = jnp.min(cand, axis=1, keepdims=True)
            jv = jnp.minimum(jv, float(NPAD - 1))
            idx_ref[:, t:t + 1] = jv.astype(jnp.int32)
            score = jnp.where(colf == jv, BIG2, score)

    @pl.when(j == NWC - 1)
    def _():
        span = sp_ref[i]

        @pl.when(span <= NWC // 2)
        def _():
            extract(NWC // 2 * CC)

        @pl.when(span > NWC // 2)
        def _():
            extract(NWC * CC)


def _knn_win(xin, xint, batch_col, batch_row, chunk_lo, span, d):
    grid_spec = pltpu.PrefetchScalarGridSpec(
        num_scalar_prefetch=2,
        grid=(NBLK, NWC),
        in_specs=[
            pl.BlockSpec((R, d), lambda i, j, cl, sp: (i, 0)),
            pl.BlockSpec((d, CC),
                         lambda i, j, cl, sp:
                         (0, jnp.minimum(cl[i] + j, NCH - 1))),
            pl.BlockSpec((R, 1), lambda i, j, cl, sp: (i, 0)),
            pl.BlockSpec((1, CC),
                         lambda i, j, cl, sp:
                         (0, jnp.minimum(cl[i] + j, NCH - 1))),
        ],
        out_specs=pl.BlockSpec((R, K), lambda i, j, cl, sp: (i, 0)),
        scratch_shapes=[pltpu.VMEM((R, NWC * CC), jnp.float32)],
    )
    return pl.pallas_call(
        _knn_win_body,
        grid_spec=grid_spec,
        out_shape=jax.ShapeDtypeStruct((NPAD, K), jnp.int32),
    )(chunk_lo, span, xin, xint, batch_col, batch_row)


def _knn(xin, xint, batch_col, batch_row, batch_p, starts, d):
    rows = jnp.arange(NBLK) * R
    g_lo = jnp.minimum(batch_p[rows], B - 1)
    g_hi = jnp.minimum(batch_p[rows + R - 1], B - 1)
    lo_col = starts[g_lo]
    hi_col = jnp.maximum(starts[g_hi + 1], lo_col + 1)
    chunk_lo = (lo_col // CC).astype(jnp.int32)
    span = ((hi_col - 1) // CC - chunk_lo + 1).astype(jnp.int32)
    fits = jnp.max(span) <= NWC
    return lax.cond(
        fits,
        lambda: _knn_win(xin, xint, batch_col, batch_row, chunk_lo, span, d),
        lambda: _knn_full(xin, xint, batch_col, batch_row, d),
    )


def _sc_gather(table, idx_flat, h):
    nk = NPAD * K
    per_w = nk // SC_NW
    chunk = 64
    n_iter = per_w // chunk
    mesh = plsc.VectorSubcoreMesh(core_axis_name="c", subcore_axis_name="s")

    n2 = n_iter // 2

    @functools.partial(
        pl.kernel,
        mesh=mesh,
        out_type=jax.ShapeDtypeStruct((nk, h), jnp.float32),
        scratch_types=[
            pltpu.VMEM((chunk,), jnp.int32),
            pltpu.VMEM((chunk,), jnp.int32),
            pltpu.VMEM((chunk, h), jnp.float32),
            pltpu.VMEM((chunk, h), jnp.float32),
            pltpu.SemaphoreType.DMA,
            pltpu.SemaphoreType.DMA,
        ],
    )
    def gk(table_hbm, idx_hbm, out_hbm, idx_a, idx_b, rows_a, rows_b,
           sem_a, sem_b):
        wid = lax.axis_index("s") * SC_NC + lax.axis_index("c")
        base = wid * per_w

        pltpu.sync_copy(idx_hbm.at[pl.ds(base, chunk)], idx_a)
        pltpu.async_copy(table_hbm.at[idx_a], rows_a, sem_a)

        def body(t, carry):
            off0 = base + (2 * t) * chunk
            off1 = off0 + chunk
            pltpu.sync_copy(idx_hbm.at[pl.ds(off1, chunk)], idx_b)
            pltpu.async_copy(table_hbm.at[idx_b], rows_b, sem_b)
            pltpu.make_async_copy(table_hbm.at[idx_a], rows_a, sem_a).wait()
            pltpu.sync_copy(rows_a, out_hbm.at[pl.ds(off0, chunk)])

            @pl.when(t < n2 - 1)
            def _():
                pltpu.sync_copy(idx_hbm.at[pl.ds(off1 + chunk, chunk)], idx_a)
                pltpu.async_copy(table_hbm.at[idx_a], rows_a, sem_a)

            pltpu.make_async_copy(table_hbm.at[idx_b], rows_b, sem_b).wait()
            pltpu.sync_copy(rows_b, out_hbm.at[pl.ds(off1, chunk)])
            return carry

        lax.fori_loop(0, n2, body, 0)

    return gk(table, idx_flat)


def _conv_body(x_ref, g_ref, bc_ref, w1_ref, b1_ref, w2_ref, b2_ref,
               o_ref, st_ref):
    xb = x_ref[...]
    w1 = w1_ref[...]
    w2 = w2_ref[...]
    b1 = b1_ref[...]
    acc = None
    for k in range(K):
        xj = g_ref[:, k, :]
        e = jnp.concatenate([xb, xj - xb], axis=1)
        h = _elu(_dotD(e, w1) + b1)
        p = _dotD(h, w2)
        acc = p if acc is None else jnp.maximum(acc, p)
    o = acc + b2_ref[...]
    o_ref[...] = o
    bc = bc_ref[...]
    m8 = (bc == lax.broadcasted_iota(jnp.int32, (1, B), 1)).astype(jnp.float32)
    cnt = jnp.sum(m8, axis=0, keepdims=True)
    s1 = jnp.sum(m8 * jnp.sum(o, axis=1, keepdims=True), axis=0, keepdims=True)
    part = jnp.concatenate([cnt, s1, jnp.zeros((6, B), jnp.float32)], axis=0)

    @pl.when(pl.program_id(0) == 0)
    def _():
        st_ref[...] = part

    @pl.when(pl.program_id(0) != 0)
    def _():
        st_ref[...] += part


def _conv(xin, g3, batch_col, w1, b1, w2, b2, d, h, o):
    return pl.pallas_call(
        _conv_body,
        grid=(NBLK,),
        in_specs=[
            pl.BlockSpec((R, d), lambda i: (i, 0)),
            pl.BlockSpec((R, K, d), lambda i: (i, 0, 0)),
            pl.BlockSpec((R, 1), lambda i: (i, 0)),
            pl.BlockSpec((2 * d, h), lambda i: (0, 0)),
            pl.BlockSpec((1, h), lambda i: (0, 0)),
            pl.BlockSpec((h, o), lambda i: (0, 0)),
            pl.BlockSpec((1, o), lambda i: (0, 0)),
        ],
        out_specs=(
            pl.BlockSpec((R, o), lambda i: (i, 0)),
            pl.BlockSpec((8, B), lambda i: (0, 0)),
        ),
        out_shape=(
            jax.ShapeDtypeStruct((NPAD, o), jnp.float32),
            jax.ShapeDtypeStruct((8, B), jnp.float32),
        ),
    )(xin, g3, batch_col, w1, b1, w2, b2)


def _stats2_body(x_ref, bc_ref, st1_ref, st2_ref):
    x = x_ref[...]
    f = x.shape[1]
    bc = bc_ref[...]
    cnt = st1_ref[0:1, :]
    s1 = st1_ref[1:2, :]
    denom = jnp.maximum(cnt, 1.0) * float(f)
    mean = s1 / denom
    m8 = (bc == lax.broadcasted_iota(jnp.int32, (1, B), 1)).astype(jnp.float32)
    mean_b = jnp.sum(m8 * mean, axis=1, keepdims=True)
    xc = x - mean_b
    s2 = jnp.sum(m8 * jnp.sum(xc * xc, axis=1, keepdims=True), axis=0,
                 keepdims=True)
    part = jnp.concatenate([s2, jnp.zeros((7, B), jnp.float32)], axis=0)

    @pl.when(pl.program_id(0) == 0)
    def _():
        st2_ref[...] = part

    @pl.when(pl.program_id(0) != 0)
    def _():
        st2_ref[...] += part


def _stats2(x, batch_col, st1, f):
    return pl.pallas_call(
        _stats2_body,
        grid=(NBLK,),
        in_specs=[
            pl.BlockSpec((R, f), lambda i: (i, 0)),
            pl.BlockSpec((R, 1), lambda i: (i, 0)),
            pl.BlockSpec((8, B), lambda i: (0, 0)),
        ],
        out_specs=pl.BlockSpec((8, B), lambda i: (0, 0)),
        out_shape=jax.ShapeDtypeStruct((8, B), jnp.float32),
    )(x, batch_col, st1)


def _ln_body(x_ref, bc_ref, st1_ref, st2_ref, w_ref, b_ref, o_ref):
    x = x_ref[...]
    f = x.shape[1]
    bc = bc_ref[...]
    cnt = st1_ref[0:1, :]
    s1 = st1_ref[1:2, :]
    s2 = st2_ref[0:1, :]
    denom = jnp.maximum(cnt, 1.0) * float(f)
    mean = s1 / denom
    var = s2 / denom
    m8 = (bc == lax.broadcasted_iota(jnp.int32, (1, B), 1)).astype(jnp.float32)
    mean_b = jnp.sum(m8 * mean, axis=1, keepdims=True)
    var_b = jnp.sum(m8 * var, axis=1, keepdims=True)
    out = (x - mean_b) * lax.rsqrt(var_b + 1e-5) * w_ref[...] + b_ref[...]
    out = _elu(out)
    o_ref[...] = jnp.where(bc < B, out, 0.0)


def _graph_ln(x, batch_col, st1, st2, w, b, f):
    return pl.pallas_call(
        _ln_body,
        grid=(NBLK,),
        in_specs=[
            pl.BlockSpec((R, f), lambda i: (i, 0)),
            pl.BlockSpec((R, 1), lambda i: (i, 0)),
            pl.BlockSpec((8, B), lambda i: (0, 0)),
            pl.BlockSpec((8, B), lambda i: (0, 0)),
            pl.BlockSpec((1, f), lambda i: (0, 0)),
            pl.BlockSpec((1, f), lambda i: (0, 0)),
        ],
        out_specs=pl.BlockSpec((R, f), lambda i: (i, 0)),
        out_shape=jax.ShapeDtypeStruct((NPAD, f), jnp.float32),
    )(x, batch_col, st1, st2, w, b)


def _head_body(x_ref, br_ref, c_ref, w1a_ref, w1b_ref, b1_ref, nw1_ref,
               nb1_ref, w2_ref, b2_ref, nw2_ref, nb2_ref, w3_ref, b3_ref,
               o_ref):
    br = br_ref[...]
    m8t = (lax.broadcasted_iota(jnp.int32, (B, 1), 0) == br).astype(
        jnp.float32)
    cnt = jnp.sum(m8t, axis=1, keepdims=True)
    pooled = _dotH(m8t, x_ref[...]) / jnp.maximum(cnt, 1.0)

    def ln(v, w, bb):
        mu = jnp.mean(v, axis=-1, keepdims=True)
        va = jnp.mean((v - mu) ** 2, axis=-1, keepdims=True)
        return (v - mu) * lax.rsqrt(va + 1e-5) * w + bb

    h = (_dotD(pooled, w1a_ref[...]) + _dotD(c_ref[...], w1b_ref[...])
         + b1_ref[...])
    h = _elu(ln(h, nw1_ref[...], nb1_ref[...]))
    h = _dotD(h, w2_ref[...]) + b2_ref[...]
    h = _elu(ln(h, nw2_ref[...], nb2_ref[...]))
    o_ref[...] = _dotD(h, w3_ref[...]) + b3_ref[...]


def _head(x, batch_row, c, fw1a, fw1b, fb1, fnw1, fnb1, fw2, fb2, fnw2,
          fnb2, fw3, fb3):
    return pl.pallas_call(
        _head_body,
        out_shape=jax.ShapeDtypeStruct((B, 1), jnp.float32),
    )(x, batch_row, c, fw1a, fw1b, fb1, fnw1, fnb1, fw2, fb2, fnw2, fnb2,
      fw3, fb3)


def kernel(pos, y, batch, emb_W, ce_W1, ce_b1, ce_W2, ce_b2,
           cw1_0, cb1_0, cw2_0, cb2_0, nw_0, nb_0,
           cw1_1, cb1_1, cw2_1, cb2_1, nw_1, nb_1,
           cw1_2, cb1_2, cw2_2, cb2_2, nw_2, nb_2,
           fw1, fb1, fnw1, fnb1, fw2, fb2, fnw2, fnb2, fw3, fb3):
    batch = batch.astype(jnp.int32)
    batch_p = jnp.pad(batch, (0, NPAD - N), constant_values=B)
    batch_col = batch_p.reshape(NPAD, 1)
    batch_row = batch_p.reshape(1, NPAD)
    y2 = y.astype(jnp.int32).reshape(B, 1)

    c, cb, starts2d = _prep(y2, batch_col, emb_W, ce_W1, ce_b1, ce_W2, ce_b2)
    starts = starts2d.reshape(16)

    hc = [3, 128, 256, 512]
    conv_params = [(cw1_0, cb1_0, cw2_0, cb2_0),
                   (cw1_1, cb1_1, cw2_1, cb2_1),
                   (cw1_2, cb1_2, cw2_2, cb2_2)]
    norm_params = [(nw_0, nb_0), (nw_1, nb_1), (nw_2, nb_2)]

    x = jnp.pad(pos, ((0, NPAD - N), (0, 0)))
    for i in range(3):
        in_c = hc[i] + 128
        hid = hc[i] * 2
        out = hc[i + 1]
        w1, b1, w2, b2 = conv_params[i]
        nw, nb = norm_params[i]
        dp = max(256, ((in_c + 127) // 128) * 128)
        hp = max(128, ((hid + 127) // 128) * 128)

        xin = jnp.concatenate([x, cb], axis=1)
        if xin.shape[1] < dp:
            xin = jnp.pad(xin, ((0, 0), (0, dp - xin.shape[1])))
        w1a = jnp.pad(w1[:in_c], ((0, dp - in_c), (0, 0)))
        w1b = jnp.pad(w1[in_c:], ((0, dp - in_c), (0, 0)))
        w1p = jnp.concatenate([w1a, w1b], axis=0)
        if hid < hp:
            w1p = jnp.pad(w1p, ((0, 0), (0, hp - hid)))
            b1p = jnp.pad(b1, (0, hp - hid))
            w2p = jnp.pad(w2, ((0, hp - hid), (0, 0)))
        else:
            b1p = b1
            w2p = w2

        idx = _knn(xin, xin.T, batch_col, batch_row, batch_p, starts, dp)
        g = _sc_gather(xin, idx.reshape(NPAD * K), dp)
        xconv, st1 = _conv(xin, g.reshape(NPAD, K, dp), batch_col, w1p,
                           b1p.reshape(1, hp), w2p, b2.reshape(1, out),
                           dp, hp, out)
        st2 = _stats2(xconv, batch_col, st1, out)
        x = _graph_ln(xconv, batch_col, st1, st2, nw.reshape(1, out),
                      nb.reshape(1, out), out)

    fw1a = fw1[:hc[-1]]
    fw1b = fw1[hc[-1]:]
    return _head(x, batch_row, c, fw1a, fw1b, fb1.reshape(1, -1),
                 fnw1.reshape(1, -1), fnb1.reshape(1, -1), fw2,
                 fb2.reshape(1, -1), fnw2.reshape(1, -1), fnb2.reshape(1, -1),
                 fw3, fb3.reshape(1, -1))


# speedup vs baseline: 10.8123x; 1.0005x over previous
import functools

import jax
import jax.numpy as jnp
from jax import lax
from jax.experimental import pallas as pl
from jax.experimental.pallas import tpu as pltpu
from jax.experimental.pallas import tpu_sc as plsc

N = 10000
NPAD = 10240
B = 8
K = 16
R = 256
NBLK = NPAD // R
BIG = 1e30
BIG2 = 2e30

SC_NC = 2
SC_NS = 16
SC_NW = SC_NC * SC_NS


def _dotH(a, b):
    return jnp.dot(a, b, precision=jax.lax.Precision.HIGHEST,
                   preferred_element_type=jnp.float32)


def _dotD(a, b):
    return jnp.dot(a, b, precision=jax.lax.Precision.DEFAULT,
                   preferred_element_type=jnp.float32)


def _elu(x):
    return jnp.where(x > 0, x, jnp.exp(jnp.minimum(x, 0.0)) - 1.0)


def _prep_body(y_ref, bc_ref, emb_ref, w1_ref, b1_ref, w2_ref, b2_ref,
               c_ref, cb_ref, st_ref):
    y = y_ref[...]
    oh = (y == lax.broadcasted_iota(jnp.int32, (1, 16), 1)).astype(jnp.float32)
    c = _elu(_dotH(oh, emb_ref[...]))
    c = _elu(_dotD(c, w1_ref[...]) + b1_ref[...])
    c = _dotD(c, w2_ref[...]) + b2_ref[...]
    c_ref[...] = c
    bc = bc_ref[...]
    m = (bc == lax.broadcasted_iota(jnp.int32, (1, B), 1)).astype(jnp.float32)
    cb_ref[...] = _dotH(m, c)
    lt = (bc < lax.broadcasted_iota(jnp.int32, (1, 16), 1)).astype(jnp.float32)
    st_ref[...] = jnp.sum(lt, axis=0, keepdims=True).astype(jnp.int32)


def _prep(y2, batch_col, emb_W, ce_W1, ce_b1, ce_W2, ce_b2):
    return pl.pallas_call(
        _prep_body,
        out_shape=(
            jax.ShapeDtypeStruct((B, 128), jnp.float32),
            jax.ShapeDtypeStruct((NPAD, 128), jnp.float32),
            jax.ShapeDtypeStruct((1, 16), jnp.int32),
        ),
    )(y2, batch_col, emb_W, ce_W1, ce_b1, ce_W2, ce_b2)


def _knn_body(x_ref, xt_ref, bc_ref, br_ref, idx_ref):
    blk = pl.program_id(0)
    xb = x_ref[...]
    xt = xt_ref[...]
    p = _dotD(xb, xt)
    sqj = jnp.sum(xt * xt, axis=0, keepdims=True)
    sqi = jnp.sum(xb * xb, axis=1, keepdims=True)
    score = (sqi - 2.0 * p) + sqj
    bc = bc_ref[...]
    br = br_ref[...]
    rowf = (blk * R + lax.broadcasted_iota(jnp.int32, (R, 1), 0)).astype(
        jnp.float32)
    colf = lax.broadcasted_iota(jnp.int32, (1, NPAD), 1).astype(jnp.float32)
    ok = (bc == br) & (bc < B) & (rowf != colf)
    score = jnp.where(ok, score, BIG)
    for t in range(K):
        m = jnp.min(score, axis=1, keepdims=True)
        cand = jnp.where(score == m, colf, float(NPAD))
        j = jnp.min(cand, axis=1, keepdims=True)
        j = jnp.minimum(j, float(NPAD - 1))
        idx_ref[:, t:t + 1] = j.astype(jnp.int32)
        score = jnp.where(colf == j, BIG2, score)


def _knn_full(xin, xint, batch_col, batch_row, d):
    return pl.pallas_call(
        _knn_body,
        grid=(NBLK,),
        in_specs=[
            pl.BlockSpec((R, d), lambda i: (i, 0)),
            pl.BlockSpec((d, NPAD), lambda i: (0, 0)),
            pl.BlockSpec((R, 1), lambda i: (i, 0)),
            pl.BlockSpec((1, NPAD), lambda i: (0, 0)),
        ],
        out_specs=pl.BlockSpec((R, K), lambda i: (i, 0)),
        out_shape=jax.ShapeDtypeStruct((NPAD, K), jnp.int32),
    )(xin, xint, batch_col, batch_row)


CC = 256
NWC = 16
NCH = NPAD // CC


def _knn_win_body(cl_ref, sp_ref, x_ref, xt_ref, bc_ref, br_ref, idx_ref,
                  sc_ref):
    i = pl.program_id(0)
    j = pl.program_id(1)
    base = cl_ref[i]
    actual = base + j
    valid = actual < NCH
    xb = x_ref[...]
    bc = bc_ref[...]

    @pl.when(valid)
    def _():
        xt = xt_ref[...]
        p = _dotD(xb, xt)
        sqj = jnp.sum(xt * xt, axis=0, keepdims=True)
        sqi = jnp.sum(xb * xb, axis=1, keepdims=True)
        score = (sqi - 2.0 * p) + sqj
        br = br_ref[...]
        rowf = (i * R + lax.broadcasted_iota(jnp.int32, (R, 1), 0)).astype(
            jnp.float32)
        colf = (actual * CC + lax.broadcasted_iota(
            jnp.int32, (1, CC), 1)).astype(jnp.float32)
        ok = (bc == br) & (bc < B) & (rowf != colf)
        sc_ref[:, pl.ds(j * CC, CC)] = jnp.where(ok, score, BIG)

    @pl.when(jnp.logical_not(valid))
    def _():
        sc_ref[:, pl.ds(j * CC, CC)] = jnp.full((R, CC), BIG, jnp.float32)

    def extract(w):
        score = sc_ref[:, :w]
        colf = (base * CC).astype(jnp.float32) + lax.broadcasted_iota(
            jnp.int32, (1, w), 1).astype(jnp.float32)
        for t in range(K):
            m = jnp.min(score, axis=1, keepdims=True)
            cand = jnp.where(score == m, colf, float(NPAD))
            jv = jnp.min(cand, axis=1, keepdims=True)
            jv = jnp.minimum(jv, float(NPAD - 1))
            idx_ref[:, t:t + 1] = jv.astype(jnp.int32)
            score = jnp.where(colf == jv, BIG2, score)

    @pl.when(j == NWC - 1)
    def _():
        span = sp_ref[i]

        @pl.when(span <= NWC // 2)
        def _():
            extract(NWC // 2 * CC)

        @pl.when(span > NWC // 2)
        def _():
            extract(NWC * CC)


def _knn_win(xin, xint, batch_col, batch_row, chunk_lo, span, d):
    grid_spec = pltpu.PrefetchScalarGridSpec(
        num_scalar_prefetch=2,
        grid=(NBLK, NWC),
        in_specs=[
            pl.BlockSpec((R, d), lambda i, j, cl, sp: (i, 0)),
            pl.BlockSpec((d, CC),
                         lambda i, j, cl, sp:
                         (0, jnp.minimum(cl[i] + j, NCH - 1))),
            pl.BlockSpec((R, 1), lambda i, j, cl, sp: (i, 0)),
            pl.BlockSpec((1, CC),
                         lambda i, j, cl, sp:
                         (0, jnp.minimum(cl[i] + j, NCH - 1))),
        ],
        out_specs=pl.BlockSpec((R, K), lambda i, j, cl, sp: (i, 0)),
        scratch_shapes=[pltpu.VMEM((R, NWC * CC), jnp.float32)],
    )
    return pl.pallas_call(
        _knn_win_body,
        grid_spec=grid_spec,
        out_shape=jax.ShapeDtypeStruct((NPAD, K), jnp.int32),
    )(chunk_lo, span, xin, xint, batch_col, batch_row)


def _knn(xin, xint, batch_col, batch_row, batch_p, starts, d):
    rows = jnp.arange(NBLK) * R
    g_lo = jnp.minimum(batch_p[rows], B - 1)
    g_hi = jnp.minimum(batch_p[rows + R - 1], B - 1)
    lo_col = starts[g_lo]
    hi_col = jnp.maximum(starts[g_hi + 1], lo_col + 1)
    chunk_lo = (lo_col // CC).astype(jnp.int32)
    span = ((hi_col - 1) // CC - chunk_lo + 1).astype(jnp.int32)
    fits = jnp.max(span) <= NWC
    return lax.cond(
        fits,
        lambda: _knn_win(xin, xint, batch_col, batch_row, chunk_lo, span, d),
        lambda: _knn_full(xin, xint, batch_col, batch_row, d),
    )


def _sc_gather(table, idx_flat, h):
    nk = NPAD * K
    per_w = nk // SC_NW
    chunk = 128
    n_iter = per_w // chunk
    mesh = plsc.VectorSubcoreMesh(core_axis_name="c", subcore_axis_name="s")

    n2 = n_iter // 2

    @functools.partial(
        pl.kernel,
        mesh=mesh,
        out_type=jax.ShapeDtypeStruct((nk, h), jnp.float32),
        scratch_types=[
            pltpu.VMEM((chunk,), jnp.int32),
            pltpu.VMEM((chunk,), jnp.int32),
            pltpu.VMEM((chunk, h), jnp.float32),
            pltpu.VMEM((chunk, h), jnp.float32),
            pltpu.SemaphoreType.DMA,
            pltpu.SemaphoreType.DMA,
        ],
    )
    def gk(table_hbm, idx_hbm, out_hbm, idx_a, idx_b, rows_a, rows_b,
           sem_a, sem_b):
        wid = lax.axis_index("s") * SC_NC + lax.axis_index("c")
        base = wid * per_w

        pltpu.sync_copy(idx_hbm.at[pl.ds(base, chunk)], idx_a)
        pltpu.async_copy(table_hbm.at[idx_a], rows_a, sem_a)

        def body(t, carry):
            off0 = base + (2 * t) * chunk
            off1 = off0 + chunk
            pltpu.sync_copy(idx_hbm.at[pl.ds(off1, chunk)], idx_b)
            pltpu.async_copy(table_hbm.at[idx_b], rows_b, sem_b)
            pltpu.make_async_copy(table_hbm.at[idx_a], rows_a, sem_a).wait()
            pltpu.sync_copy(rows_a, out_hbm.at[pl.ds(off0, chunk)])

            @pl.when(t < n2 - 1)
            def _():
                pltpu.sync_copy(idx_hbm.at[pl.ds(off1 + chunk, chunk)], idx_a)
                pltpu.async_copy(table_hbm.at[idx_a], rows_a, sem_a)

            pltpu.make_async_copy(table_hbm.at[idx_b], rows_b, sem_b).wait()
            pltpu.sync_copy(rows_b, out_hbm.at[pl.ds(off1, chunk)])
            return carry

        lax.fori_loop(0, n2, body, 0)

    return gk(table, idx_flat)


def _conv_body(x_ref, g_ref, bc_ref, w1_ref, b1_ref, w2_ref, b2_ref,
               o_ref, st_ref):
    xb = x_ref[...]
    w1 = w1_ref[...]
    w2 = w2_ref[...]
    b1 = b1_ref[...]
    acc = None
    for k in range(K):
        xj = g_ref[:, k, :]
        e = jnp.concatenate([xb, xj - xb], axis=1)
        h = _elu(_dotD(e, w1) + b1)
        p = _dotD(h, w2)
        acc = p if acc is None else jnp.maximum(acc, p)
    o = acc + b2_ref[...]
    o_ref[...] = o
    bc = bc_ref[...]
    m8 = (bc == lax.broadcasted_iota(jnp.int32, (1, B), 1)).astype(jnp.float32)
    cnt = jnp.sum(m8, axis=0, keepdims=True)
    s1 = jnp.sum(m8 * jnp.sum(o, axis=1, keepdims=True), axis=0, keepdims=True)
    part = jnp.concatenate([cnt, s1, jnp.zeros((6, B), jnp.float32)], axis=0)

    @pl.when(pl.program_id(0) == 0)
    def _():
        st_ref[...] = part

    @pl.when(pl.program_id(0) != 0)
    def _():
        st_ref[...] += part


def _conv(xin, g3, batch_col, w1, b1, w2, b2, d, h, o):
    return pl.pallas_call(
        _conv_body,
        grid=(NBLK,),
        in_specs=[
            pl.BlockSpec((R, d), lambda i: (i, 0)),
            pl.BlockSpec((R, K, d), lambda i: (i, 0, 0)),
            pl.BlockSpec((R, 1), lambda i: (i, 0)),
            pl.BlockSpec((2 * d, h), lambda i: (0, 0)),
            pl.BlockSpec((1, h), lambda i: (0, 0)),
            pl.BlockSpec((h, o), lambda i: (0, 0)),
            pl.BlockSpec((1, o), lambda i: (0, 0)),
        ],
        out_specs=(
            pl.BlockSpec((R, o), lambda i: (i, 0)),
            pl.BlockSpec((8, B), lambda i: (0, 0)),
        ),
        out_shape=(
            jax.ShapeDtypeStruct((NPAD, o), jnp.float32),
            jax.ShapeDtypeStruct((8, B), jnp.float32),
        ),
    )(xin, g3, batch_col, w1, b1, w2, b2)


def _stats2_body(x_ref, bc_ref, st1_ref, st2_ref):
    x = x_ref[...]
    f = x.shape[1]
    bc = bc_ref[...]
    cnt = st1_ref[0:1, :]
    s1 = st1_ref[1:2, :]
    denom = jnp.maximum(cnt, 1.0) * float(f)
    mean = s1 / denom
    m8 = (bc == lax.broadcasted_iota(jnp.int32, (1, B), 1)).astype(jnp.float32)
    mean_b = jnp.sum(m8 * mean, axis=1, keepdims=True)
    xc = x - mean_b
    s2 = jnp.sum(m8 * jnp.sum(xc * xc, axis=1, keepdims=True), axis=0,
                 keepdims=True)
    part = jnp.concatenate([s2, jnp.zeros((7, B), jnp.float32)], axis=0)

    @pl.when(pl.program_id(0) == 0)
    def _():
        st2_ref[...] = part

    @pl.when(pl.program_id(0) != 0)
    def _():
        st2_ref[...] += part


def _stats2(x, batch_col, st1, f):
    return pl.pallas_call(
        _stats2_body,
        grid=(NBLK,),
        in_specs=[
            pl.BlockSpec((R, f), lambda i: (i, 0)),
            pl.BlockSpec((R, 1), lambda i: (i, 0)),
            pl.BlockSpec((8, B), lambda i: (0, 0)),
        ],
        out_specs=pl.BlockSpec((8, B), lambda i: (0, 0)),
        out_shape=jax.ShapeDtypeStruct((8, B), jnp.float32),
    )(x, batch_col, st1)


def _ln_body(x_ref, bc_ref, st1_ref, st2_ref, w_ref, b_ref, o_ref):
    x = x_ref[...]
    f = x.shape[1]
    bc = bc_ref[...]
    cnt = st1_ref[0:1, :]
    s1 = st1_ref[1:2, :]
    s2 = st2_ref[0:1, :]
    denom = jnp.maximum(cnt, 1.0) * float(f)
    mean = s1 / denom
    var = s2 / denom
    m8 = (bc == lax.broadcasted_iota(jnp.int32, (1, B), 1)).astype(jnp.float32)
    mean_b = jnp.sum(m8 * mean, axis=1, keepdims=True)
    var_b = jnp.sum(m8 * var, axis=1, keepdims=True)
    out = (x - mean_b) * lax.rsqrt(var_b + 1e-5) * w_ref[...] + b_ref[...]
    out = _elu(out)
    o_ref[...] = jnp.where(bc < B, out, 0.0)


def _graph_ln(x, batch_col, st1, st2, w, b, f):
    return pl.pallas_call(
        _ln_body,
        grid=(NBLK,),
        in_specs=[
            pl.BlockSpec((R, f), lambda i: (i, 0)),
            pl.BlockSpec((R, 1), lambda i: (i, 0)),
            pl.BlockSpec((8, B), lambda i: (0, 0)),
            pl.BlockSpec((8, B), lambda i: (0, 0)),
            pl.BlockSpec((1, f), lambda i: (0, 0)),
            pl.BlockSpec((1, f), lambda i: (0, 0)),
        ],
        out_specs=pl.BlockSpec((R, f), lambda i: (i, 0)),
        out_shape=jax.ShapeDtypeStruct((NPAD, f), jnp.float32),
    )(x, batch_col, st1, st2, w, b)


def _head_body(x_ref, br_ref, c_ref, w1a_ref, w1b_ref, b1_ref, nw1_ref,
               nb1_ref, w2_ref, b2_ref, nw2_ref, nb2_ref, w3_ref, b3_ref,
               o_ref):
    br = br_ref[...]
    m8t = (lax.broadcasted_iota(jnp.int32, (B, 1), 0) == br).astype(
        jnp.float32)
    cnt = jnp.sum(m8t, axis=1, keepdims=True)
    pooled = _dotH(m8t, x_ref[...]) / jnp.maximum(cnt, 1.0)

    def ln(v, w, bb):
        mu = jnp.mean(v, axis=-1, keepdims=True)
        va = jnp.mean((v - mu) ** 2, axis=-1, keepdims=True)
        return (v - mu) * lax.rsqrt(va + 1e-5) * w + bb

    h = (_dotD(pooled, w1a_ref[...]) + _dotD(c_ref[...], w1b_ref[...])
         + b1_ref[...])
    h = _elu(ln(h, nw1_ref[...], nb1_ref[...]))
    h = _dotD(h, w2_ref[...]) + b2_ref[...]
    h = _elu(ln(h, nw2_ref[...], nb2_ref[...]))
    o_ref[...] = _dotD(h, w3_ref[...]) + b3_ref[...]


def _head(x, batch_row, c, fw1a, fw1b, fb1, fnw1, fnb1, fw2, fb2, fnw2,
          fnb2, fw3, fb3):
    return pl.pallas_call(
        _head_body,
        out_shape=jax.ShapeDtypeStruct((B, 1), jnp.float32),
    )(x, batch_row, c, fw1a, fw1b, fb1, fnw1, fnb1, fw2, fb2, fnw2, fnb2,
      fw3, fb3)


def kernel(pos, y, batch, emb_W, ce_W1, ce_b1, ce_W2, ce_b2,
           cw1_0, cb1_0, cw2_0, cb2_0, nw_0, nb_0,
           cw1_1, cb1_1, cw2_1, cb2_1, nw_1, nb_1,
           cw1_2, cb1_2, cw2_2, cb2_2, nw_2, nb_2,
           fw1, fb1, fnw1, fnb1, fw2, fb2, fnw2, fnb2, fw3, fb3):
    batch = batch.astype(jnp.int32)
    batch_p = jnp.pad(batch, (0, NPAD - N), constant_values=B)
    batch_col = batch_p.reshape(NPAD, 1)
    batch_row = batch_p.reshape(1, NPAD)
    y2 = y.astype(jnp.int32).reshape(B, 1)

    c, cb, starts2d = _prep(y2, batch_col, emb_W, ce_W1, ce_b1, ce_W2, ce_b2)
    starts = starts2d.reshape(16)

    hc = [3, 128, 256, 512]
    conv_params = [(cw1_0, cb1_0, cw2_0, cb2_0),
                   (cw1_1, cb1_1, cw2_1, cb2_1),
                   (cw1_2, cb1_2, cw2_2, cb2_2)]
    norm_params = [(nw_0, nb_0), (nw_1, nb_1), (nw_2, nb_2)]

    x = jnp.pad(pos, ((0, NPAD - N), (0, 0)))
    for i in range(3):
        in_c = hc[i] + 128
        hid = hc[i] * 2
        out = hc[i + 1]
        w1, b1, w2, b2 = conv_params[i]
        nw, nb = norm_params[i]
        dp = max(256, ((in_c + 127) // 128) * 128)
        hp = max(128, ((hid + 127) // 128) * 128)

        xin = jnp.concatenate([x, cb], axis=1)
        if xin.shape[1] < dp:
            xin = jnp.pad(xin, ((0, 0), (0, dp - xin.shape[1])))
        w1a = jnp.pad(w1[:in_c], ((0, dp - in_c), (0, 0)))
        w1b = jnp.pad(w1[in_c:], ((0, dp - in_c), (0, 0)))
        w1p = jnp.concatenate([w1a, w1b], axis=0)
        if hid < hp:
            w1p = jnp.pad(w1p, ((0, 0), (0, hp - hid)))
            b1p = jnp.pad(b1, (0, hp - hid))
            w2p = jnp.pad(w2, ((0, hp - hid), (0, 0)))
        else:
            b1p = b1
            w2p = w2

        idx = _knn(xin, xin.T, batch_col, batch_row, batch_p, starts, dp)
        g = _sc_gather(xin, idx.reshape(NPAD * K), dp)
        xconv, st1 = _conv(xin, g.reshape(NPAD, K, dp), batch_col, w1p,
                           b1p.reshape(1, hp), w2p, b2.reshape(1, out),
                           dp, hp, out)
        st2 = _stats2(xconv, batch_col, st1, out)
        x = _graph_ln(xconv, batch_col, st1, st2, nw.reshape(1, out),
                      nb.reshape(1, out), out)

    fw1a = fw1[:hc[-1]]
    fw1b = fw1[hc[-1]:]
    return _head(x, batch_row, c, fw1a, fw1b, fb1.reshape(1, -1),
                 fnw1.reshape(1, -1), fnb1.reshape(1, -1), fw2,
                 fb2.reshape(1, -1), fnw2.reshape(1, -1), fnb2.reshape(1, -1),
                 fw3, fb3.reshape(1, -1))


# speedup vs baseline: 11.1383x; 1.0302x over previous
import functools

import jax
import jax.numpy as jnp
from jax import lax
from jax.experimental import pallas as pl
from jax.experimental.pallas import tpu as pltpu
from jax.experimental.pallas import tpu_sc as plsc

N = 10000
NPAD = 10240
B = 8
K = 16
R = 256
NBLK = NPAD // R
BIG = 1e30
BIG2 = 2e30

SC_NC = 2
SC_NS = 16
SC_NW = SC_NC * SC_NS


def _dotH(a, b):
    return jnp.dot(a, b, precision=jax.lax.Precision.HIGHEST,
                   preferred_element_type=jnp.float32)


def _dotD(a, b):
    return jnp.dot(a, b, precision=jax.lax.Precision.DEFAULT,
                   preferred_element_type=jnp.float32)


def _elu(x):
    return jnp.where(x > 0, x, jnp.exp(jnp.minimum(x, 0.0)) - 1.0)


def _prep_body(y_ref, bc_ref, emb_ref, w1_ref, b1_ref, w2_ref, b2_ref,
               c_ref, cb_ref, st_ref):
    y = y_ref[...]
    oh = (y == lax.broadcasted_iota(jnp.int32, (1, 16), 1)).astype(jnp.float32)
    c = _elu(_dotH(oh, emb_ref[...]))
    c = _elu(_dotD(c, w1_ref[...]) + b1_ref[...])
    c = _dotD(c, w2_ref[...]) + b2_ref[...]
    c_ref[...] = c
    bc = bc_ref[...]
    m = (bc == lax.broadcasted_iota(jnp.int32, (1, B), 1)).astype(jnp.float32)
    cb_ref[...] = _dotH(m, c)
    lt = (bc < lax.broadcasted_iota(jnp.int32, (1, 16), 1)).astype(jnp.float32)
    st_ref[...] = jnp.sum(lt, axis=0, keepdims=True).astype(jnp.int32)


def _prep(y2, batch_col, emb_W, ce_W1, ce_b1, ce_W2, ce_b2):
    return pl.pallas_call(
        _prep_body,
        out_shape=(
            jax.ShapeDtypeStruct((B, 128), jnp.float32),
            jax.ShapeDtypeStruct((NPAD, 128), jnp.float32),
            jax.ShapeDtypeStruct((1, 16), jnp.int32),
        ),
    )(y2, batch_col, emb_W, ce_W1, ce_b1, ce_W2, ce_b2)


def _knn_body(x_ref, xt_ref, bc_ref, br_ref, idx_ref):
    blk = pl.program_id(0)
    xb = x_ref[...]
    xt = xt_ref[...]
    p = _dotD(xb, xt)
    sqj = jnp.sum(xt * xt, axis=0, keepdims=True)
    sqi = jnp.sum(xb * xb, axis=1, keepdims=True)
    score = (sqi - 2.0 * p) + sqj
    bc = bc_ref[...]
    br = br_ref[...]
    rowf = (blk * R + lax.broadcasted_iota(jnp.int32, (R, 1), 0)).astype(
        jnp.float32)
    colf = lax.broadcasted_iota(jnp.int32, (1, NPAD), 1).astype(jnp.float32)
    ok = (bc == br) & (bc < B) & (rowf != colf)
    score = jnp.where(ok, score, BIG)
    for t in range(K):
        m = jnp.min(score, axis=1, keepdims=True)
        cand = jnp.where(score == m, colf, float(NPAD))
        j = jnp.min(cand, axis=1, keepdims=True)
        j = jnp.minimum(j, float(NPAD - 1))
        idx_ref[:, t:t + 1] = j.astype(jnp.int32)
        score = jnp.where(colf == j, BIG2, score)


def _knn_full(xin, xint, batch_col, batch_row, d):
    return pl.pallas_call(
        _knn_body,
        grid=(NBLK,),
        in_specs=[
            pl.BlockSpec((R, d), lambda i: (i, 0)),
            pl.BlockSpec((d, NPAD), lambda i: (0, 0)),
            pl.BlockSpec((R, 1), lambda i: (i, 0)),
            pl.BlockSpec((1, NPAD), lambda i: (0, 0)),
        ],
        out_specs=pl.BlockSpec((R, K), lambda i: (i, 0)),
        out_shape=jax.ShapeDtypeStruct((NPAD, K), jnp.int32),
    )(xin, xint, batch_col, batch_row)


CC = 256
NWC = 16
NCH = NPAD // CC


def _knn_win_body(cl_ref, sp_ref, x_ref, xt_ref, bc_ref, br_ref, idx_ref,
                  sc_ref):
    i = pl.program_id(0)
    j = pl.program_id(1)
    base = cl_ref[i]
    actual = base + j
    valid = actual < NCH
    xb = x_ref[...]
    bc = bc_ref[...]

    @pl.when(valid)
    def _():
        xt = xt_ref[...]
        p = _dotD(xb, xt)
        sqj = jnp.sum(xt * xt, axis=0, keepdims=True)
        sqi = jnp.sum(xb * xb, axis=1, keepdims=True)
        score = (sqi - 2.0 * p) + sqj
        br = br_ref[...]
        rowf = (i * R + lax.broadcasted_iota(jnp.int32, (R, 1), 0)).astype(
            jnp.float32)
        colf = (actual * CC + lax.broadcasted_iota(
            jnp.int32, (1, CC), 1)).astype(jnp.float32)
        ok = (bc == br) & (bc < B) & (rowf != colf)
        sc_ref[:, pl.ds(j * CC, CC)] = jnp.where(ok, score, BIG)

    @pl.when(jnp.logical_not(valid))
    def _():
        sc_ref[:, pl.ds(j * CC, CC)] = jnp.full((R, CC), BIG, jnp.float32)

    def extract(w):
        score = sc_ref[:, :w]
        colf = (base * CC).astype(jnp.float32) + lax.broadcasted_iota(
            jnp.int32, (1, w), 1).astype(jnp.float32)
        for t in range(K):
            m = jnp.min(score, axis=1, keepdims=True)
            cand = jnp.where(score == m, colf, float(NPAD))
            jv = jnp.min(cand, axis=1, keepdims=True)
            jv = jnp.minimum(jv, float(NPAD - 1))
            idx_ref[:, t:t + 1] = jv.astype(jnp.int32)
            score = jnp.where(colf == jv, BIG2, score)

    @pl.when(j == NWC - 1)
    def _():
        span = sp_ref[i]

        @pl.when(span <= NWC // 2)
        def _():
            extract(NWC // 2 * CC)

        @pl.when(span > NWC // 2)
        def _():
            extract(NWC * CC)


def _knn_win(xin, xint, batch_col, batch_row, chunk_lo, span, d):
    grid_spec = pltpu.PrefetchScalarGridSpec(
        num_scalar_prefetch=2,
        grid=(NBLK, NWC),
        in_specs=[
            pl.BlockSpec((R, d), lambda i, j, cl, sp: (i, 0)),
            pl.BlockSpec((d, CC),
                         lambda i, j, cl, sp:
                         (0, jnp.minimum(cl[i] + j, NCH - 1))),
            pl.BlockSpec((R, 1), lambda i, j, cl, sp: (i, 0)),
            pl.BlockSpec((1, CC),
                         lambda i, j, cl, sp:
                         (0, jnp.minimum(cl[i] + j, NCH - 1))),
        ],
        out_specs=pl.BlockSpec((R, K), lambda i, j, cl, sp: (i, 0)),
        scratch_shapes=[pltpu.VMEM((R, NWC * CC), jnp.float32)],
    )
    return pl.pallas_call(
        _knn_win_body,
        grid_spec=grid_spec,
        out_shape=jax.ShapeDtypeStruct((NPAD, K), jnp.int32),
    )(chunk_lo, span, xin, xint, batch_col, batch_row)


def _knn(xin, xint, batch_col, batch_row, batch_p, starts, d):
    rows = jnp.arange(NBLK) * R
    g_lo = jnp.minimum(batch_p[rows], B - 1)
    g_hi = jnp.minimum(batch_p[rows + R - 1], B - 1)
    lo_col = starts[g_lo]
    hi_col = jnp.maximum(starts[g_hi + 1], lo_col + 1)
    chunk_lo = (lo_col // CC).astype(jnp.int32)
    span = ((hi_col - 1) // CC - chunk_lo + 1).astype(jnp.int32)
    fits = jnp.max(span) <= NWC
    return lax.cond(
        fits,
        lambda: _knn_win(xin, xint, batch_col, batch_row, chunk_lo, span, d),
        lambda: _knn_full(xin, xint, batch_col, batch_row, d),
    )


def _sc_gather(table, idx_flat, h):
    nk = idx_flat.shape[0]
    per_w = nk // SC_NW
    chunk = 128
    n_iter = per_w // chunk
    mesh = plsc.VectorSubcoreMesh(core_axis_name="c", subcore_axis_name="s")

    n2 = n_iter // 2

    @functools.partial(
        pl.kernel,
        mesh=mesh,
        out_type=jax.ShapeDtypeStruct((nk, h), jnp.float32),
        scratch_types=[
            pltpu.VMEM((chunk,), jnp.int32),
            pltpu.VMEM((chunk,), jnp.int32),
            pltpu.VMEM((chunk, h), jnp.float32),
            pltpu.VMEM((chunk, h), jnp.float32),
            pltpu.SemaphoreType.DMA,
            pltpu.SemaphoreType.DMA,
        ],
    )
    def gk(table_hbm, idx_hbm, out_hbm, idx_a, idx_b, rows_a, rows_b,
           sem_a, sem_b):
        wid = lax.axis_index("s") * SC_NC + lax.axis_index("c")
        base = wid * per_w

        pltpu.sync_copy(idx_hbm.at[pl.ds(base, chunk)], idx_a)
        pltpu.async_copy(table_hbm.at[idx_a], rows_a, sem_a)

        def body(t, carry):
            off0 = base + (2 * t) * chunk
            off1 = off0 + chunk
            pltpu.sync_copy(idx_hbm.at[pl.ds(off1, chunk)], idx_b)
            pltpu.async_copy(table_hbm.at[idx_b], rows_b, sem_b)
            pltpu.make_async_copy(table_hbm.at[idx_a], rows_a, sem_a).wait()
            pltpu.sync_copy(rows_a, out_hbm.at[pl.ds(off0, chunk)])

            @pl.when(t < n2 - 1)
            def _():
                pltpu.sync_copy(idx_hbm.at[pl.ds(off1 + chunk, chunk)], idx_a)
                pltpu.async_copy(table_hbm.at[idx_a], rows_a, sem_a)

            pltpu.make_async_copy(table_hbm.at[idx_b], rows_b, sem_b).wait()
            pltpu.sync_copy(rows_b, out_hbm.at[pl.ds(off1, chunk)])
            return carry

        lax.fori_loop(0, n2, body, 0)

    return gk(table, idx_flat)


def _conv_body(x_ref, g_ref, bc_ref, w1_ref, b1_ref, w2_ref, b2_ref,
               o_ref, st_ref):
    xb = x_ref[...]
    w1 = w1_ref[...]
    w2 = w2_ref[...]
    b1 = b1_ref[...]
    acc = None
    for k in range(K):
        xj = g_ref[:, k, :]
        e = jnp.concatenate([xb, xj - xb], axis=1)
        h = _elu(_dotD(e, w1) + b1)
        p = _dotD(h, w2)
        acc = p if acc is None else jnp.maximum(acc, p)
    o = acc + b2_ref[...]
    o_ref[...] = o
    bc = bc_ref[...]
    m8 = (bc == lax.broadcasted_iota(jnp.int32, (1, B), 1)).astype(jnp.float32)
    cnt = jnp.sum(m8, axis=0, keepdims=True)
    s1 = jnp.sum(m8 * jnp.sum(o, axis=1, keepdims=True), axis=0, keepdims=True)
    part = jnp.concatenate([cnt, s1, jnp.zeros((6, B), jnp.float32)], axis=0)

    @pl.when(pl.program_id(0) == 0)
    def _():
        st_ref[...] = part

    @pl.when(pl.program_id(0) != 0)
    def _():
        st_ref[...] += part


def _conv(xin, g3, batch_col, w1, b1, w2, b2, d, h, o, off, nb):
    return pl.pallas_call(
        _conv_body,
        grid=(nb,),
        in_specs=[
            pl.BlockSpec((R, d), lambda i: (off + i, 0)),
            pl.BlockSpec((R, K, d), lambda i: (i, 0, 0)),
            pl.BlockSpec((R, 1), lambda i: (off + i, 0)),
            pl.BlockSpec((2 * d, h), lambda i: (0, 0)),
            pl.BlockSpec((1, h), lambda i: (0, 0)),
            pl.BlockSpec((h, o), lambda i: (0, 0)),
            pl.BlockSpec((1, o), lambda i: (0, 0)),
        ],
        out_specs=(
            pl.BlockSpec((R, o), lambda i: (i, 0)),
            pl.BlockSpec((8, B), lambda i: (0, 0)),
        ),
        out_shape=(
            jax.ShapeDtypeStruct((nb * R, o), jnp.float32),
            jax.ShapeDtypeStruct((8, B), jnp.float32),
        ),
    )(xin, g3, batch_col, w1, b1, w2, b2)


def _stats2_body(x_ref, bc_ref, st1_ref, st2_ref):
    x = x_ref[...]
    f = x.shape[1]
    bc = bc_ref[...]
    cnt = st1_ref[0:1, :]
    s1 = st1_ref[1:2, :]
    denom = jnp.maximum(cnt, 1.0) * float(f)
    mean = s1 / denom
    m8 = (bc == lax.broadcasted_iota(jnp.int32, (1, B), 1)).astype(jnp.float32)
    mean_b = jnp.sum(m8 * mean, axis=1, keepdims=True)
    xc = x - mean_b
    s2 = jnp.sum(m8 * jnp.sum(xc * xc, axis=1, keepdims=True), axis=0,
                 keepdims=True)
    part = jnp.concatenate([s2, jnp.zeros((7, B), jnp.float32)], axis=0)

    @pl.when(pl.program_id(0) == 0)
    def _():
        st2_ref[...] = part

    @pl.when(pl.program_id(0) != 0)
    def _():
        st2_ref[...] += part


def _stats2(x, batch_col, st1, f):
    return pl.pallas_call(
        _stats2_body,
        grid=(NBLK,),
        in_specs=[
            pl.BlockSpec((R, f), lambda i: (i, 0)),
            pl.BlockSpec((R, 1), lambda i: (i, 0)),
            pl.BlockSpec((8, B), lambda i: (0, 0)),
        ],
        out_specs=pl.BlockSpec((8, B), lambda i: (0, 0)),
        out_shape=jax.ShapeDtypeStruct((8, B), jnp.float32),
    )(x, batch_col, st1)


def _ln_body(x_ref, bc_ref, st1_ref, st2_ref, w_ref, b_ref, o_ref):
    x = x_ref[...]
    f = x.shape[1]
    bc = bc_ref[...]
    cnt = st1_ref[0:1, :]
    s1 = st1_ref[1:2, :]
    s2 = st2_ref[0:1, :]
    denom = jnp.maximum(cnt, 1.0) * float(f)
    mean = s1 / denom
    var = s2 / denom
    m8 = (bc == lax.broadcasted_iota(jnp.int32, (1, B), 1)).astype(jnp.float32)
    mean_b = jnp.sum(m8 * mean, axis=1, keepdims=True)
    var_b = jnp.sum(m8 * var, axis=1, keepdims=True)
    out = (x - mean_b) * lax.rsqrt(var_b + 1e-5) * w_ref[...] + b_ref[...]
    out = _elu(out)
    o_ref[...] = jnp.where(bc < B, out, 0.0)


def _graph_ln(x, batch_col, st1, st2, w, b, f):
    return pl.pallas_call(
        _ln_body,
        grid=(NBLK,),
        in_specs=[
            pl.BlockSpec((R, f), lambda i: (i, 0)),
            pl.BlockSpec((R, 1), lambda i: (i, 0)),
            pl.BlockSpec((8, B), lambda i: (0, 0)),
            pl.BlockSpec((8, B), lambda i: (0, 0)),
            pl.BlockSpec((1, f), lambda i: (0, 0)),
            pl.BlockSpec((1, f), lambda i: (0, 0)),
        ],
        out_specs=pl.BlockSpec((R, f), lambda i: (i, 0)),
        out_shape=jax.ShapeDtypeStruct((NPAD, f), jnp.float32),
    )(x, batch_col, st1, st2, w, b)


def _head_body(x_ref, br_ref, c_ref, w1a_ref, w1b_ref, b1_ref, nw1_ref,
               nb1_ref, w2_ref, b2_ref, nw2_ref, nb2_ref, w3_ref, b3_ref,
               o_ref):
    br = br_ref[...]
    m8t = (lax.broadcasted_iota(jnp.int32, (B, 1), 0) == br).astype(
        jnp.float32)
    cnt = jnp.sum(m8t, axis=1, keepdims=True)
    pooled = _dotH(m8t, x_ref[...]) / jnp.maximum(cnt, 1.0)

    def ln(v, w, bb):
        mu = jnp.mean(v, axis=-1, keepdims=True)
        va = jnp.mean((v - mu) ** 2, axis=-1, keepdims=True)
        return (v - mu) * lax.rsqrt(va + 1e-5) * w + bb

    h = (_dotD(pooled, w1a_ref[...]) + _dotD(c_ref[...], w1b_ref[...])
         + b1_ref[...])
    h = _elu(ln(h, nw1_ref[...], nb1_ref[...]))
    h = _dotD(h, w2_ref[...]) + b2_ref[...]
    h = _elu(ln(h, nw2_ref[...], nb2_ref[...]))
    o_ref[...] = _dotD(h, w3_ref[...]) + b3_ref[...]


def _head(x, batch_row, c, fw1a, fw1b, fb1, fnw1, fnb1, fw2, fb2, fnw2,
          fnb2, fw3, fb3):
    return pl.pallas_call(
        _head_body,
        out_shape=jax.ShapeDtypeStruct((B, 1), jnp.float32),
    )(x, batch_row, c, fw1a, fw1b, fb1, fnw1, fnb1, fw2, fb2, fnw2, fnb2,
      fw3, fb3)


def kernel(pos, y, batch, emb_W, ce_W1, ce_b1, ce_W2, ce_b2,
           cw1_0, cb1_0, cw2_0, cb2_0, nw_0, nb_0,
           cw1_1, cb1_1, cw2_1, cb2_1, nw_1, nb_1,
           cw1_2, cb1_2, cw2_2, cb2_2, nw_2, nb_2,
           fw1, fb1, fnw1, fnb1, fw2, fb2, fnw2, fnb2, fw3, fb3):
    batch = batch.astype(jnp.int32)
    batch_p = jnp.pad(batch, (0, NPAD - N), constant_values=B)
    batch_col = batch_p.reshape(NPAD, 1)
    batch_row = batch_p.reshape(1, NPAD)
    y2 = y.astype(jnp.int32).reshape(B, 1)

    c, cb, starts2d = _prep(y2, batch_col, emb_W, ce_W1, ce_b1, ce_W2, ce_b2)
    starts = starts2d.reshape(16)

    hc = [3, 128, 256, 512]
    conv_params = [(cw1_0, cb1_0, cw2_0, cb2_0),
                   (cw1_1, cb1_1, cw2_1, cb2_1),
                   (cw1_2, cb1_2, cw2_2, cb2_2)]
    norm_params = [(nw_0, nb_0), (nw_1, nb_1), (nw_2, nb_2)]

    x = jnp.pad(pos, ((0, NPAD - N), (0, 0)))
    for i in range(3):
        in_c = hc[i] + 128
        hid = hc[i] * 2
        out = hc[i + 1]
        w1, b1, w2, b2 = conv_params[i]
        nw, nb = norm_params[i]
        dp = max(256, ((in_c + 127) // 128) * 128)
        hp = max(128, ((hid + 127) // 128) * 128)

        xin = jnp.concatenate([x, cb], axis=1)
        if xin.shape[1] < dp:
            xin = jnp.pad(xin, ((0, 0), (0, dp - xin.shape[1])))
        w1a = jnp.pad(w1[:in_c], ((0, dp - in_c), (0, 0)))
        w1b = jnp.pad(w1[in_c:], ((0, dp - in_c), (0, 0)))
        w1p = jnp.concatenate([w1a, w1b], axis=0)
        if hid < hp:
            w1p = jnp.pad(w1p, ((0, 0), (0, hp - hid)))
            b1p = jnp.pad(b1, (0, hp - hid))
            w2p = jnp.pad(w2, ((0, hp - hid), (0, 0)))
        else:
            b1p = b1
            w2p = w2

        idx = _knn(xin, xin.T, batch_col, batch_row, batch_p, starts, dp)
        idx_flat = idx.reshape(NPAD * K)
        half = NPAD * K // 2
        nh = NBLK // 2
        g0 = _sc_gather(xin, idx_flat[:half], dp)
        g1 = _sc_gather(xin, idx_flat[half:], dp)
        xc0, st1a = _conv(xin, g0.reshape(NPAD // 2, K, dp), batch_col, w1p,
                          b1p.reshape(1, hp), w2p, b2.reshape(1, out),
                          dp, hp, out, 0, nh)
        xc1, st1b = _conv(xin, g1.reshape(NPAD // 2, K, dp), batch_col, w1p,
                          b1p.reshape(1, hp), w2p, b2.reshape(1, out),
                          dp, hp, out, nh, nh)
        xconv = jnp.concatenate([xc0, xc1], axis=0)
        st1 = st1a + st1b
        st2 = _stats2(xconv, batch_col, st1, out)
        x = _graph_ln(xconv, batch_col, st1, st2, nw.reshape(1, out),
                      nb.reshape(1, out), out)

    fw1a = fw1[:hc[-1]]
    fw1b = fw1[hc[-1]:]
    return _head(x, batch_row, c, fw1a, fw1b, fb1.reshape(1, -1),
                 fnw1.reshape(1, -1), fnb1.reshape(1, -1), fw2,
                 fb2.reshape(1, -1), fnw2.reshape(1, -1), fnb2.reshape(1, -1),
                 fw3, fb3.reshape(1, -1))
